# Initial kernel scaffold; baseline (speedup 1.0000x reference)
import jax
import jax.numpy as jnp
from jax import lax
from jax.experimental import pallas as pl
from jax.experimental.pallas import tpu as pltpu
from jax.experimental.pallas import tpu_sc as plsc

N = 10000
D = 128
R = 8
NB = 4
E = 320000
EPS = 1e-3
INV_BN = 1.0 / float(jnp.sqrt(jnp.float32(1.0 + EPS)))

NCORES = 2
NSUB = 16
NTILES = NCORES * NSUB
CK = 128
NCH = 80
ET = NCH * CK
E_PAD = NTILES * ET
N_ACC = 10240
ROWS_PER_TILE = N_ACC // NSUB
CW = 16
BN_BLK = 1000
GRID = N // BN_BLK


def _make_sc_pass(with_counts):
    mesh = plsc.VectorSubcoreMesh(core_axis_name="c", subcore_axis_name="s")
    out_type = [jax.ShapeDtypeStruct((NCORES, N_ACC, D), jnp.float32)]
    if with_counts:
        out_type.append(jax.ShapeDtypeStruct((NCORES, N_ACC, CW), jnp.float32))
    scratch = [
        pltpu.VMEM((NCH, CK), jnp.int32),
        pltpu.VMEM((NCH, CK), jnp.int32),
        pltpu.VMEM((NCH, CK), jnp.int32),
        pltpu.VMEM((NCH, CK), jnp.int32),
        pltpu.VMEM((CK, D), jnp.float32),
        pltpu.VMEM_SHARED((N_ACC, D), jnp.float32),
    ]
    if with_counts:
        scratch.append(pltpu.VMEM((CK, CW), jnp.float32))
        scratch.append(pltpu.VMEM_SHARED((N_ACC, CW), jnp.float32))

    def body(proj_hbm, src_hbm, dst_hbm, typ_hbm, *rest):
        if with_counts:
            (acc_out, cnt_out, src_b, typ_b, dst_b, key_b, rows_b, acc_s,
             oh_b, cnt_s) = rest
        else:
            acc_out, src_b, typ_b, dst_b, key_b, rows_b, acc_s = rest
        c = lax.axis_index("c")
        s = lax.axis_index("s")
        wid = c * NSUB + s

        pltpu.sync_copy(src_hbm.at[pl.ds(wid * NCH, NCH)], src_b)
        pltpu.sync_copy(dst_hbm.at[pl.ds(wid * NCH, NCH)], dst_b)
        pltpu.sync_copy(typ_hbm.at[pl.ds(wid * NCH, NCH)], typ_b)

        def zrow(i, _):
            for v in range(D // 16):
                rows_b[i, pl.ds(v * 16, 16)] = jnp.zeros((16,), jnp.float32)
            return 0
        lax.fori_loop(0, CK, zrow, 0)
        for blk in range(ROWS_PER_TILE // CK):
            pltpu.sync_copy(
                rows_b, acc_s.at[pl.ds(s * ROWS_PER_TILE + blk * CK, CK)])
        if with_counts:
            def zoh(i, _):
                oh_b[i, pl.ds(0, CW)] = jnp.zeros((CW,), jnp.float32)
                return 0
            lax.fori_loop(0, CK, zoh, 0)
            for blk in range(ROWS_PER_TILE // CK):
                pltpu.sync_copy(
                    oh_b, cnt_s.at[pl.ds(s * ROWS_PER_TILE + blk * CK, CK)])

        def kbody(i, _):
            row = i // (CK // 16)
            seg = (i % (CK // 16)) * 16
            tv = typ_b[row, pl.ds(seg, 16)]
            sv = src_b[row, pl.ds(seg, 16)]
            key_b[row, pl.ds(seg, 16)] = tv * N + sv
            return 0
        lax.fori_loop(0, NCH * (CK // 16), kbody, 0)

        plsc.subcore_barrier()

        ones16 = jnp.ones((16,), jnp.float32)
        zeros16 = jnp.zeros((16,), jnp.float32)
        iota16 = lax.iota(jnp.int32, 16)

        def chunk(j, _):
            pltpu.sync_copy(proj_hbm.at[key_b.at[j]], rows_b)
            pltpu.sync_copy(rows_b, acc_s.at[dst_b.at[j]], add=True)
            if with_counts:
                for v in range(CK // 16):
                    tv = typ_b[j, pl.ds(v * 16, 16)]
                    plsc.store_scatter(oh_b, [iota16 + (v * 16), tv], ones16)
                pltpu.sync_copy(oh_b, cnt_s.at[dst_b.at[j]], add=True)
                for v in range(CK // 16):
                    tv = typ_b[j, pl.ds(v * 16, 16)]
                    plsc.store_scatter(oh_b, [iota16 + (v * 16), tv], zeros16)
            return 0
        lax.fori_loop(0, NCH, chunk, 0)

        plsc.subcore_barrier()

        pltpu.sync_copy(
            acc_s.at[pl.ds(s * ROWS_PER_TILE, ROWS_PER_TILE)],
            acc_out.at[c, pl.ds(s * ROWS_PER_TILE, ROWS_PER_TILE)])
        if with_counts:
            pltpu.sync_copy(
                cnt_s.at[pl.ds(s * ROWS_PER_TILE, ROWS_PER_TILE)],
                cnt_out.at[c, pl.ds(s * ROWS_PER_TILE, ROWS_PER_TILE)])

    return pl.kernel(
        body,
        out_type=tuple(out_type) if with_counts else out_type[0],
        mesh=mesh,
        scratch_types=scratch,
    )


_sc_pass_counts = _make_sc_pass(True)
_sc_pass_plain = _make_sc_pass(False)


def _make_dense(with_emb):
    def body(*refs):
        if with_emb:
            x_ref, ew_ref, eb_ref, bs_ref, cf_ref, lp_ref, proj_ref, hl_ref = refs
            h = jnp.dot(x_ref[...], ew_ref[...],
                        preferred_element_type=jnp.float32) + eb_ref[...]
        else:
            x_ref, bs_ref, cf_ref, lp_ref, proj_ref, hl_ref = refs
            h = x_ref[...]
        hl_ref[...] = jnp.dot(h, lp_ref[...], preferred_element_type=jnp.float32)
        for b in range(NB):
            t = jnp.dot(h, bs_ref[b], preferred_element_type=jnp.float32)
            for r in range(R):
                contrib = cf_ref[r, b] * t
                if b == 0:
                    proj_ref[r] = contrib
                else:
                    proj_ref[r] = proj_ref[r] + contrib

    in_specs = [pl.BlockSpec((BN_BLK, D), lambda i: (i, 0))]
    if with_emb:
        in_specs += [
            pl.BlockSpec((D, D), lambda i: (0, 0)),
            pl.BlockSpec((1, D), lambda i: (0, 0)),
        ]
    in_specs += [
        pl.BlockSpec((NB, D, D), lambda i: (0, 0, 0)),
        pl.BlockSpec(memory_space=pltpu.SMEM),
        pl.BlockSpec((D, D), lambda i: (0, 0)),
    ]
    return pl.pallas_call(
        body,
        grid=(GRID,),
        in_specs=in_specs,
        out_specs=(
            pl.BlockSpec((R, BN_BLK, D), lambda i: (0, i, 0)),
            pl.BlockSpec((BN_BLK, D), lambda i: (i, 0)),
        ),
        out_shape=(
            jax.ShapeDtypeStruct((R, N, D), jnp.float32),
            jax.ShapeDtypeStruct((N, D), jnp.float32),
        ),
    )


_dense_emb = _make_dense(True)
_dense_plain = _make_dense(False)


def _finalize_body(acc_ref, cnt_ref, hl_ref, g_ref, b_ref, o_ref):
    p = acc_ref[0] + acc_ref[1]
    cn = cnt_ref[0] + cnt_ref[1]
    norm = jnp.zeros((BN_BLK, 1), jnp.float32)
    for r_ in range(R):
        cr = cn[:, r_:r_ + 1]
        norm = jnp.where(cr > 0, 1.0 / cr, norm)
    o = p * norm + hl_ref[...]
    o = g_ref[...] * (o * INV_BN) + b_ref[...]
    o_ref[...] = jnp.maximum(o, 0.0)


_finalize = pl.pallas_call(
    _finalize_body,
    grid=(GRID,),
    in_specs=[
        pl.BlockSpec((NCORES, BN_BLK, D), lambda i: (0, i, 0)),
        pl.BlockSpec((NCORES, BN_BLK, CW), lambda i: (0, i, 0)),
        pl.BlockSpec((BN_BLK, D), lambda i: (i, 0)),
        pl.BlockSpec((1, D), lambda i: (0, 0)),
        pl.BlockSpec((1, D), lambda i: (0, 0)),
    ],
    out_specs=pl.BlockSpec((BN_BLK, D), lambda i: (i, 0)),
    out_shape=jax.ShapeDtypeStruct((N, D), jnp.float32),
)


def kernel(x, edge_index, edge_type, emb_w, emb_b, basis0, coef0, loop0,
           gamma0, beta0, basis1, coef1, loop1, gamma1, beta1):
    src = edge_index[0].astype(jnp.int32)
    dst = edge_index[1].astype(jnp.int32)
    typ = edge_type.astype(jnp.int32)

    npad = E_PAD - E
    ar = jnp.arange(npad, dtype=jnp.int32)
    src_p = jnp.concatenate([src, ar % N]).reshape(E_PAD // CK, CK)
    dst_p = jnp.concatenate([dst, N + ar % (N_ACC - N)]).reshape(E_PAD // CK, CK)
    typ_p = jnp.concatenate([typ, jnp.zeros((npad,), jnp.int32)]).reshape(
        E_PAD // CK, CK)

    emb_b2 = emb_b.reshape(1, D)
    g0 = gamma0.reshape(1, D)
    b0 = beta0.reshape(1, D)
    g1 = gamma1.reshape(1, D)
    b1 = beta1.reshape(1, D)

    proj1, hl1 = _dense_emb(x, emb_w, emb_b2, basis0, coef0, loop0)
    acc1, cnt = _sc_pass_counts(proj1.reshape(R * N, D), src_p, dst_p, typ_p)
    h1 = _finalize(acc1, cnt, hl1, g0, b0)

    proj2, hl2 = _dense_plain(h1, basis1, coef1, loop1)
    acc2 = _sc_pass_plain(proj2.reshape(R * N, D), src_p, dst_p, typ_p)
    h2 = _finalize(acc2, cnt, hl2, g1, b1)
    return h2


# speedup vs baseline: 21.8618x; 21.8618x over previous
import math

import numpy as np

import jax
import jax.numpy as jnp
from jax import lax
from jax.experimental import pallas as pl
from jax.experimental.pallas import tpu as pltpu
from jax.experimental.pallas import tpu_sc as plsc

N = 10000
D = 128
R = 8
NB = 4
E = 320000
EPS = 1e-3
INV_BN = float(1.0 / math.sqrt(1.0 + EPS))

NCORES = 2
NSUB = 16
NTILES = NCORES * NSUB
CK = 128
NCH = 80
ET = NCH * CK
E_PAD = NTILES * ET
N_ACC = 10240
ROWS_PER_TILE = N_ACC // NSUB
CW = 16
BN_BLK = 1000
GRID = N // BN_BLK

_I0 = np.int32(0)
_ICK = np.int32(CK)
_ICH = np.int32(NCH)
_IN = np.int32(N)
_INSUB = np.int32(NSUB)
_IRPT = np.int32(ROWS_PER_TILE)
_I32F = np.int32(32)
_I31 = np.int32(31)

_OHTAB = np.repeat(np.eye(CW, dtype=np.float32)[:R], 32, axis=0)


def _make_sc_pass(with_counts):
    mesh = plsc.VectorSubcoreMesh(core_axis_name="c", subcore_axis_name="s")
    out_type = [jax.ShapeDtypeStruct((NCORES, N_ACC, D), jnp.float32)]
    if with_counts:
        out_type.append(jax.ShapeDtypeStruct((NCORES, N_ACC, CW), jnp.float32))
    scratch = [
        pltpu.VMEM((1, CK), jnp.int32),
        pltpu.VMEM((1, CK), jnp.int32),
        pltpu.VMEM((1, CK), jnp.int32),
        pltpu.VMEM((1, CK), jnp.int32),
        pltpu.VMEM((CK, D), jnp.float32),
        pltpu.VMEM_SHARED((N_ACC, D), jnp.float32),
    ]
    if with_counts:
        scratch.append(pltpu.VMEM((CK, CW), jnp.float32))
        scratch.append(pltpu.VMEM((1, CK), jnp.int32))
        scratch.append(pltpu.VMEM_SHARED((N_ACC, CW), jnp.float32))

    def body(proj_hbm, src_hbm, dst_hbm, typ_hbm, *rest):
        if with_counts:
            (ohtab_hbm, acc_out, cnt_out, src_b, typ_b, dst_b, key_b, rows_b,
             acc_s, oh_b, key2_b, cnt_s) = rest
        else:
            acc_out, src_b, typ_b, dst_b, key_b, rows_b, acc_s = rest
        c = lax.axis_index("c")
        s = lax.axis_index("s")
        wid = c * _INSUB + s
        rbase = s * _IRPT

        def zrow(i, _):
            for v in range(D // 16):
                rows_b[i, pl.ds(v * 16, 16)] = jnp.zeros((16,), jnp.float32)
            return _I0
        lax.fori_loop(_I0, _ICK, zrow, _I0)
        for blk in range(ROWS_PER_TILE // CK):
            pltpu.sync_copy(rows_b, acc_s.at[pl.ds(rbase + np.int32(blk * CK), CK)])
        if with_counts:
            def zoh(i, _):
                oh_b[i, pl.ds(0, CW)] = jnp.zeros((CW,), jnp.float32)
                return _I0
            lax.fori_loop(_I0, _ICK, zoh, _I0)
            for blk in range(ROWS_PER_TILE // CK):
                pltpu.sync_copy(
                    oh_b, cnt_s.at[pl.ds(rbase + np.int32(blk * CK), CK)])

        plsc.subcore_barrier()

        def chunk(j, _):
            jj = lax.convert_element_type(j, jnp.int32)
            row = wid * _ICH + jj
            pltpu.sync_copy(src_hbm.at[pl.ds(row, 1)], src_b)
            pltpu.sync_copy(dst_hbm.at[pl.ds(row, 1)], dst_b)
            pltpu.sync_copy(typ_hbm.at[pl.ds(row, 1)], typ_b)
            for v in range(CK // 16):
                tv = typ_b[0, pl.ds(v * 16, 16)]
                sv = src_b[0, pl.ds(v * 16, 16)]
                key_b[0, pl.ds(v * 16, 16)] = tv * _IN + sv
                if with_counts:
                    key2_b[0, pl.ds(v * 16, 16)] = tv * _I32F + (sv & _I31)
            pltpu.sync_copy(proj_hbm.at[key_b.at[0]], rows_b)
            pltpu.sync_copy(rows_b, acc_s.at[dst_b.at[0]], add=True)
            if with_counts:
                pltpu.sync_copy(ohtab_hbm.at[key2_b.at[0]], oh_b)
                pltpu.sync_copy(oh_b, cnt_s.at[dst_b.at[0]], add=True)
            return _I0
        lax.fori_loop(_I0, _ICH, chunk, _I0)

        plsc.subcore_barrier()

        pltpu.sync_copy(
            acc_s.at[pl.ds(rbase, ROWS_PER_TILE)],
            acc_out.at[c, pl.ds(rbase, ROWS_PER_TILE)])
        if with_counts:
            pltpu.sync_copy(
                cnt_s.at[pl.ds(rbase, ROWS_PER_TILE)],
                cnt_out.at[c, pl.ds(rbase, ROWS_PER_TILE)])

    return pl.kernel(
        body,
        out_type=tuple(out_type) if with_counts else out_type[0],
        mesh=mesh,
        compiler_params=pltpu.CompilerParams(
            needs_layout_passes=False, use_tc_tiling_on_sc=False),
        scratch_types=scratch,
    )


_sc_pass_counts = _make_sc_pass(True)
_sc_pass_plain = _make_sc_pass(False)


def _make_dense(with_emb):
    def body(*refs):
        if with_emb:
            x_ref, ew_ref, eb_ref, bs_ref, cf_ref, lp_ref, proj_ref, hl_ref = refs
            h = jnp.dot(x_ref[...], ew_ref[...],
                        preferred_element_type=jnp.float32) + eb_ref[...]
        else:
            x_ref, bs_ref, cf_ref, lp_ref, proj_ref, hl_ref = refs
            h = x_ref[...]
        hl_ref[...] = jnp.dot(h, lp_ref[...], preferred_element_type=jnp.float32)
        for b in range(NB):
            t = jnp.dot(h, bs_ref[b], preferred_element_type=jnp.float32)
            for r in range(R):
                contrib = cf_ref[r, b] * t
                if b == 0:
                    proj_ref[r] = contrib
                else:
                    proj_ref[r] = proj_ref[r] + contrib

    in_specs = [pl.BlockSpec((BN_BLK, D), lambda i: (i, 0))]
    if with_emb:
        in_specs += [
            pl.BlockSpec((D, D), lambda i: (0, 0)),
            pl.BlockSpec((1, D), lambda i: (0, 0)),
        ]
    in_specs += [
        pl.BlockSpec((NB, D, D), lambda i: (0, 0, 0)),
        pl.BlockSpec(memory_space=pltpu.SMEM),
        pl.BlockSpec((D, D), lambda i: (0, 0)),
    ]
    return pl.pallas_call(
        body,
        grid=(GRID,),
        in_specs=in_specs,
        out_specs=(
            pl.BlockSpec((R, BN_BLK, D), lambda i: (0, i, 0)),
            pl.BlockSpec((BN_BLK, D), lambda i: (i, 0)),
        ),
        out_shape=(
            jax.ShapeDtypeStruct((R, N, D), jnp.float32),
            jax.ShapeDtypeStruct((N, D), jnp.float32),
        ),
    )


_dense_emb = _make_dense(True)
_dense_plain = _make_dense(False)


def _finalize_body(acc_ref, cnt_ref, hl_ref, g_ref, b_ref, o_ref):
    p = acc_ref[0] + acc_ref[1]
    cn = cnt_ref[0] + cnt_ref[1]
    norm = jnp.zeros((BN_BLK, 1), jnp.float32)
    for r_ in range(R):
        cr = cn[:, r_:r_ + 1]
        norm = jnp.where(cr > 0, 1.0 / cr, norm)
    o = p * norm + hl_ref[...]
    o = g_ref[...] * (o * INV_BN) + b_ref[...]
    o_ref[...] = jnp.maximum(o, 0.0)


_finalize = pl.pallas_call(
    _finalize_body,
    grid=(GRID,),
    in_specs=[
        pl.BlockSpec((NCORES, BN_BLK, D), lambda i: (0, i, 0)),
        pl.BlockSpec((NCORES, BN_BLK, CW), lambda i: (0, i, 0)),
        pl.BlockSpec((BN_BLK, D), lambda i: (i, 0)),
        pl.BlockSpec((1, D), lambda i: (0, 0)),
        pl.BlockSpec((1, D), lambda i: (0, 0)),
    ],
    out_specs=pl.BlockSpec((BN_BLK, D), lambda i: (i, 0)),
    out_shape=jax.ShapeDtypeStruct((N, D), jnp.float32),
)


def kernel(x, edge_index, edge_type, emb_w, emb_b, basis0, coef0, loop0,
           gamma0, beta0, basis1, coef1, loop1, gamma1, beta1):
    with jax.enable_x64(False):
        return _kernel_32(x, edge_index, edge_type, emb_w, emb_b, basis0,
                          coef0, loop0, gamma0, beta0, basis1, coef1, loop1,
                          gamma1, beta1)


def _kernel_32(x, edge_index, edge_type, emb_w, emb_b, basis0, coef0, loop0,
               gamma0, beta0, basis1, coef1, loop1, gamma1, beta1):
    src = edge_index[0].astype(jnp.int32)
    dst = edge_index[1].astype(jnp.int32)
    typ = edge_type.astype(jnp.int32)

    npad = E_PAD - E
    ar = jnp.arange(npad, dtype=jnp.int32)
    src_p = jnp.concatenate([src, ar % N]).reshape(E_PAD // CK, CK)
    dst_p = jnp.concatenate([dst, N + ar % (N_ACC - N)]).reshape(E_PAD // CK, CK)
    typ_p = jnp.concatenate([typ, jnp.zeros((npad,), jnp.int32)]).reshape(
        E_PAD // CK, CK)

    emb_b2 = emb_b.reshape(1, D)
    g0 = gamma0.reshape(1, D)
    b0 = beta0.reshape(1, D)
    g1 = gamma1.reshape(1, D)
    b1 = beta1.reshape(1, D)

    proj1, hl1 = _dense_emb(x, emb_w, emb_b2, basis0, coef0, loop0)
    acc1, cnt = _sc_pass_counts(proj1.reshape(R * N, D), src_p, dst_p, typ_p,
                                jnp.asarray(_OHTAB))
    h1 = _finalize(acc1, cnt, hl1, g0, b0)

    proj2, hl2 = _dense_plain(h1, basis1, coef1, loop1)
    acc2 = _sc_pass_plain(proj2.reshape(R * N, D), src_p, dst_p, typ_p)
    h2 = _finalize(acc2, cnt, hl2, g1, b1)
    return h2


# speedup vs baseline: 40.6932x; 1.8614x over previous
import math

import numpy as np

import jax
import jax.numpy as jnp
from jax import lax
from jax.experimental import pallas as pl
from jax.experimental.pallas import tpu as pltpu
from jax.experimental.pallas import tpu_sc as plsc

N = 10000
D = 128
R = 8
NB = 4
E = 320000
EPS = 1e-3
INV_BN = float(1.0 / math.sqrt(1.0 + EPS))

NCORES = 2
NSUB = 16
NTILES = NCORES * NSUB
CK = 128
NCH = 80
ET = NCH * CK
E_PAD = NTILES * ET
N_ACC = 10240
ROWS_PER_TILE = N_ACC // NSUB
CW = 16
BN_BLK = 1000
GRID = N // BN_BLK

_I0 = np.int32(0)
_ICK = np.int32(CK)
_ICH = np.int32(NCH)
_IN = np.int32(N)
_INSUB = np.int32(NSUB)
_IRPT = np.int32(ROWS_PER_TILE)
_I32F = np.int32(32)
_I31 = np.int32(31)

_OHTAB = np.repeat(np.eye(CW, dtype=np.float32)[:R], 32, axis=0)


def _make_sc_pass(with_counts):
    mesh = plsc.VectorSubcoreMesh(core_axis_name="c", subcore_axis_name="s")
    out_type = [jax.ShapeDtypeStruct((NCORES, N_ACC, D), jnp.float32)]
    if with_counts:
        out_type.append(jax.ShapeDtypeStruct((NCORES, N_ACC, CW), jnp.float32))
    scratch = [
        pltpu.VMEM((1, 3, CK), jnp.int32),
        pltpu.VMEM((2, CK), jnp.int32),
        pltpu.VMEM((2, CK), jnp.int32),
        pltpu.VMEM((2, CK, D), jnp.float32),
        pltpu.VMEM_SHARED((N_ACC, D), jnp.float32),
        pltpu.SemaphoreType.DMA((2,)),
        pltpu.SemaphoreType.DMA((2,)),
    ]
    if with_counts:
        scratch += [
            pltpu.VMEM((2, CK), jnp.int32),
            pltpu.VMEM((2, CK, CW), jnp.float32),
            pltpu.VMEM_SHARED((N_ACC, CW), jnp.float32),
            pltpu.VMEM_SHARED((R * 32, CW), jnp.float32),
            pltpu.SemaphoreType.DMA((2,)),
            pltpu.SemaphoreType.DMA((2,)),
        ]

    def body(proj_hbm, edges_hbm, *rest):
        if with_counts:
            (ohtab_hbm, acc_out, cnt_out, ebuf, dst_b, key_b, rows_b, acc_s,
             gsem, ssem, key2_b, oh_b, cnt_s, ohtab_s, ogsem, ossem) = rest
        else:
            acc_out, ebuf, dst_b, key_b, rows_b, acc_s, gsem, ssem = rest
        c = lax.axis_index("c")
        s = lax.axis_index("s")
        wid = c * _INSUB + s
        rbase = s * _IRPT

        def zrow(i, _):
            for v in range(D // 16):
                rows_b[0, i, pl.ds(v * 16, 16)] = jnp.zeros((16,), jnp.float32)
            return _I0
        lax.fori_loop(_I0, _ICK, zrow, _I0)
        for blk in range(ROWS_PER_TILE // CK):
            pltpu.sync_copy(rows_b.at[0],
                            acc_s.at[pl.ds(rbase + np.int32(blk * CK), CK)])
        if with_counts:
            def zoh(i, _):
                oh_b[0, i, pl.ds(0, CW)] = jnp.zeros((CW,), jnp.float32)
                return _I0
            lax.fori_loop(_I0, _ICK, zoh, _I0)
            for blk in range(ROWS_PER_TILE // CK):
                pltpu.sync_copy(
                    oh_b.at[0], cnt_s.at[pl.ds(rbase + np.int32(blk * CK), CK)])
            pltpu.sync_copy(ohtab_hbm, ohtab_s)

        plsc.subcore_barrier()

        def stage(j, q):
            row = wid * _ICH + j
            pltpu.sync_copy(edges_hbm.at[pl.ds(row, 1)], ebuf)
            for v in range(CK // 16):
                sv = ebuf[0, 0, pl.ds(v * 16, 16)]
                dv = ebuf[0, 1, pl.ds(v * 16, 16)]
                tv = ebuf[0, 2, pl.ds(v * 16, 16)]
                key_b[q, pl.ds(v * 16, 16)] = tv * _IN + sv
                dst_b[q, pl.ds(v * 16, 16)] = dv
                if with_counts:
                    key2_b[q, pl.ds(v * 16, 16)] = tv * _I32F + (sv & _I31)
            pltpu.async_copy(proj_hbm.at[key_b.at[q]], rows_b.at[q], gsem.at[q])
            if with_counts:
                pltpu.async_copy(ohtab_s.at[key2_b.at[q]], oh_b.at[q],
                                 ogsem.at[q])

        def wait_gather(q):
            pltpu.make_async_copy(proj_hbm.at[key_b.at[q]], rows_b.at[q],
                                  gsem.at[q]).wait()

        def wait_scatter(q):
            pltpu.make_async_copy(rows_b.at[q], acc_s.at[dst_b.at[q]],
                                  ssem.at[q]).wait()

        def wait_ohgather(q):
            pltpu.make_async_copy(ohtab_s.at[key2_b.at[q]], oh_b.at[q],
                                  ogsem.at[q]).wait()

        def wait_ohscatter(q):
            pltpu.make_async_copy(oh_b.at[q], cnt_s.at[dst_b.at[q]],
                                  ossem.at[q]).wait()

        stage(_I0, 0)
        stage(np.int32(1), 1)

        def iter2(g, _):
            for q in (0, 1):
                j = g * np.int32(2) + np.int32(q)
                wait_gather(q)
                pltpu.async_copy(rows_b.at[q], acc_s.at[dst_b.at[q]],
                                 ssem.at[q], add=True)
                if with_counts:
                    wait_ohgather(q)
                    pltpu.async_copy(oh_b.at[q], cnt_s.at[dst_b.at[q]],
                                     ossem.at[q], add=True)
                wait_scatter(q)
                if with_counts:
                    wait_ohscatter(q)
                stage(j + np.int32(2), q)
            return _I0
        lax.fori_loop(_I0, np.int32(NCH // 2), iter2, _I0)

        for q in (0, 1):
            wait_gather(q)
            if with_counts:
                wait_ohgather(q)

        plsc.subcore_barrier()

        pltpu.sync_copy(
            acc_s.at[pl.ds(rbase, ROWS_PER_TILE)],
            acc_out.at[c, pl.ds(rbase, ROWS_PER_TILE)])
        if with_counts:
            pltpu.sync_copy(
                cnt_s.at[pl.ds(rbase, ROWS_PER_TILE)],
                cnt_out.at[c, pl.ds(rbase, ROWS_PER_TILE)])

    return pl.kernel(
        body,
        out_type=tuple(out_type) if with_counts else out_type[0],
        mesh=mesh,
        compiler_params=pltpu.CompilerParams(
            needs_layout_passes=False, use_tc_tiling_on_sc=False),
        scratch_types=scratch,
    )


_sc_pass_counts = _make_sc_pass(True)
_sc_pass_plain = _make_sc_pass(False)


def _make_dense(with_emb):
    def body(*refs):
        if with_emb:
            x_ref, ew_ref, eb_ref, bs_ref, cf_ref, lp_ref, proj_ref, hl_ref = refs
            h = jnp.dot(x_ref[...], ew_ref[...],
                        preferred_element_type=jnp.float32) + eb_ref[...]
        else:
            x_ref, bs_ref, cf_ref, lp_ref, proj_ref, hl_ref = refs
            h = x_ref[...]
        hl_ref[...] = jnp.dot(h, lp_ref[...], preferred_element_type=jnp.float32)
        for b in range(NB):
            t = jnp.dot(h, bs_ref[b], preferred_element_type=jnp.float32)
            for r in range(R):
                contrib = cf_ref[r, b] * t
                if b == 0:
                    proj_ref[r] = contrib
                else:
                    proj_ref[r] = proj_ref[r] + contrib

    in_specs = [pl.BlockSpec((BN_BLK, D), lambda i: (i, 0))]
    if with_emb:
        in_specs += [
            pl.BlockSpec((D, D), lambda i: (0, 0)),
            pl.BlockSpec((1, D), lambda i: (0, 0)),
        ]
    in_specs += [
        pl.BlockSpec((NB, D, D), lambda i: (0, 0, 0)),
        pl.BlockSpec(memory_space=pltpu.SMEM),
        pl.BlockSpec((D, D), lambda i: (0, 0)),
    ]
    return pl.pallas_call(
        body,
        grid=(GRID,),
        in_specs=in_specs,
        out_specs=(
            pl.BlockSpec((R, BN_BLK, D), lambda i: (0, i, 0)),
            pl.BlockSpec((BN_BLK, D), lambda i: (i, 0)),
        ),
        out_shape=(
            jax.ShapeDtypeStruct((R, N, D), jnp.float32),
            jax.ShapeDtypeStruct((N, D), jnp.float32),
        ),
    )


_dense_emb = _make_dense(True)
_dense_plain = _make_dense(False)


def _finalize_body(acc_ref, cnt_ref, hl_ref, g_ref, b_ref, o_ref):
    p = acc_ref[0] + acc_ref[1]
    cn = cnt_ref[0] + cnt_ref[1]
    norm = jnp.zeros((BN_BLK, 1), jnp.float32)
    for r_ in range(R):
        cr = cn[:, r_:r_ + 1]
        norm = jnp.where(cr > 0, 1.0 / cr, norm)
    o = p * norm + hl_ref[...]
    o = g_ref[...] * (o * INV_BN) + b_ref[...]
    o_ref[...] = jnp.maximum(o, 0.0)


_finalize = pl.pallas_call(
    _finalize_body,
    grid=(GRID,),
    in_specs=[
        pl.BlockSpec((NCORES, BN_BLK, D), lambda i: (0, i, 0)),
        pl.BlockSpec((NCORES, BN_BLK, CW), lambda i: (0, i, 0)),
        pl.BlockSpec((BN_BLK, D), lambda i: (i, 0)),
        pl.BlockSpec((1, D), lambda i: (0, 0)),
        pl.BlockSpec((1, D), lambda i: (0, 0)),
    ],
    out_specs=pl.BlockSpec((BN_BLK, D), lambda i: (i, 0)),
    out_shape=jax.ShapeDtypeStruct((N, D), jnp.float32),
)


def kernel(x, edge_index, edge_type, emb_w, emb_b, basis0, coef0, loop0,
           gamma0, beta0, basis1, coef1, loop1, gamma1, beta1):
    with jax.enable_x64(False):
        return _kernel_32(x, edge_index, edge_type, emb_w, emb_b, basis0,
                          coef0, loop0, gamma0, beta0, basis1, coef1, loop1,
                          gamma1, beta1)


def _kernel_32(x, edge_index, edge_type, emb_w, emb_b, basis0, coef0, loop0,
               gamma0, beta0, basis1, coef1, loop1, gamma1, beta1):
    src = edge_index[0].astype(jnp.int32)
    dst = edge_index[1].astype(jnp.int32)
    typ = edge_type.astype(jnp.int32)

    npad = E_PAD - E
    ar = jnp.arange(npad, dtype=jnp.int32)
    src_p = jnp.concatenate([src, ar % N]).reshape(E_PAD // CK, CK)
    dst_p = jnp.concatenate([dst, N + ar % (N_ACC - N)]).reshape(E_PAD // CK, CK)
    typ_p = jnp.concatenate([typ, jnp.zeros((npad,), jnp.int32)]).reshape(
        E_PAD // CK, CK)
    edges = jnp.concatenate(
        [jnp.stack([src_p, dst_p, typ_p], axis=1),
         jnp.zeros((2, 3, CK), jnp.int32)], axis=0)

    emb_b2 = emb_b.reshape(1, D)
    g0 = gamma0.reshape(1, D)
    b0 = beta0.reshape(1, D)
    g1 = gamma1.reshape(1, D)
    b1 = beta1.reshape(1, D)

    proj1, hl1 = _dense_emb(x, emb_w, emb_b2, basis0, coef0, loop0)
    acc1, cnt = _sc_pass_counts(proj1.reshape(R * N, D), edges,
                                jnp.asarray(_OHTAB))
    h1 = _finalize(acc1, cnt, hl1, g0, b0)

    proj2, hl2 = _dense_plain(h1, basis1, coef1, loop1)
    acc2 = _sc_pass_plain(proj2.reshape(R * N, D), edges)
    h2 = _finalize(acc2, cnt, hl2, g1, b1)
    return h2


# speedup vs baseline: 41.3312x; 1.0157x over previous
import math

import numpy as np

import jax
import jax.numpy as jnp
from jax import lax
from jax.experimental import pallas as pl
from jax.experimental.pallas import tpu as pltpu
from jax.experimental.pallas import tpu_sc as plsc

N = 10000
D = 128
R = 8
NB = 4
E = 320000
EPS = 1e-3
INV_BN = float(1.0 / math.sqrt(1.0 + EPS))

NCORES = 2
NSUB = 16
NTILES = NCORES * NSUB
CK = 128
NCH = 80
ET = NCH * CK
E_PAD = NTILES * ET
N_ACC = 10240
ROWS_PER_TILE = N_ACC // NSUB
CW = 16
BN_BLK = 1000
GRID = N // BN_BLK

_I0 = np.int32(0)
_ICK = np.int32(CK)
_ICH = np.int32(NCH)
_IN = np.int32(N)
_INSUB = np.int32(NSUB)
_IRPT = np.int32(ROWS_PER_TILE)
_I32F = np.int32(32)
_I31 = np.int32(31)

_OHTAB = np.repeat(np.eye(CW, dtype=np.float32)[:R], 32, axis=0)


def _make_sc_pass(with_counts):
    mesh = plsc.VectorSubcoreMesh(core_axis_name="c", subcore_axis_name="s")
    out_type = [jax.ShapeDtypeStruct((NCORES, N_ACC, D), jnp.float32)]
    if with_counts:
        out_type.append(jax.ShapeDtypeStruct((NCORES, N_ACC, CW), jnp.float32))
    scratch = [
        pltpu.VMEM((1, 3, CK), jnp.int32),
        pltpu.VMEM((2, CK), jnp.int32),
        pltpu.VMEM((2, CK), jnp.int32),
        pltpu.VMEM((2, CK, D), jnp.float32),
        pltpu.VMEM_SHARED((N_ACC, D), jnp.float32),
        pltpu.SemaphoreType.DMA((2,)),
        pltpu.SemaphoreType.DMA((2,)),
    ]
    if with_counts:
        scratch += [
            pltpu.VMEM((2, CK), jnp.int32),
            pltpu.VMEM((2, CK, CW), jnp.float32),
            pltpu.VMEM_SHARED((N_ACC, CW), jnp.float32),
            pltpu.VMEM_SHARED((R * 32, CW), jnp.float32),
            pltpu.SemaphoreType.DMA((2,)),
            pltpu.SemaphoreType.DMA((2,)),
        ]

    def body(proj_hbm, edges_hbm, *rest):
        if with_counts:
            (ohtab_hbm, acc_out, cnt_out, ebuf, dst_b, key_b, rows_b, acc_s,
             gsem, ssem, key2_b, oh_b, cnt_s, ohtab_s, ogsem, ossem) = rest
        else:
            acc_out, ebuf, dst_b, key_b, rows_b, acc_s, gsem, ssem = rest
        c = lax.axis_index("c")
        s = lax.axis_index("s")
        wid = c * _INSUB + s
        rbase = s * _IRPT

        def zrow(i, _):
            for v in range(D // 16):
                rows_b[0, i, pl.ds(v * 16, 16)] = jnp.zeros((16,), jnp.float32)
            return _I0
        lax.fori_loop(_I0, _ICK, zrow, _I0)
        for blk in range(ROWS_PER_TILE // CK):
            pltpu.sync_copy(rows_b.at[0],
                            acc_s.at[pl.ds(rbase + np.int32(blk * CK), CK)])
        if with_counts:
            def zoh(i, _):
                oh_b[0, i, pl.ds(0, CW)] = jnp.zeros((CW,), jnp.float32)
                return _I0
            lax.fori_loop(_I0, _ICK, zoh, _I0)
            for blk in range(ROWS_PER_TILE // CK):
                pltpu.sync_copy(
                    oh_b.at[0], cnt_s.at[pl.ds(rbase + np.int32(blk * CK), CK)])
            pltpu.sync_copy(ohtab_hbm, ohtab_s)

        plsc.subcore_barrier()

        def stage(j, q):
            row = wid * _ICH + j
            pltpu.sync_copy(edges_hbm.at[pl.ds(row, 1)], ebuf)
            for v in range(CK // 16):
                sv = ebuf[0, 0, pl.ds(v * 16, 16)]
                dv = ebuf[0, 1, pl.ds(v * 16, 16)]
                tv = ebuf[0, 2, pl.ds(v * 16, 16)]
                key_b[q, pl.ds(v * 16, 16)] = tv * _IN + sv
                dst_b[q, pl.ds(v * 16, 16)] = dv
                if with_counts:
                    key2_b[q, pl.ds(v * 16, 16)] = tv * _I32F + (sv & _I31)
            pltpu.async_copy(proj_hbm.at[key_b.at[q]], rows_b.at[q], gsem.at[q])
            if with_counts:
                pltpu.async_copy(ohtab_s.at[key2_b.at[q]], oh_b.at[q],
                                 ogsem.at[q])

        def wait_gather(q):
            pltpu.make_async_copy(proj_hbm.at[key_b.at[q]], rows_b.at[q],
                                  gsem.at[q]).wait()

        def wait_scatter(q):
            pltpu.make_async_copy(rows_b.at[q], acc_s.at[dst_b.at[q]],
                                  ssem.at[q]).wait()

        def wait_ohgather(q):
            pltpu.make_async_copy(ohtab_s.at[key2_b.at[q]], oh_b.at[q],
                                  ogsem.at[q]).wait()

        def wait_ohscatter(q):
            pltpu.make_async_copy(oh_b.at[q], cnt_s.at[dst_b.at[q]],
                                  ossem.at[q]).wait()

        stage(_I0, 0)
        stage(np.int32(1), 1)

        def iter2(g, _):
            for q in (0, 1):
                j = g * np.int32(2) + np.int32(q)
                wait_gather(q)
                pltpu.async_copy(rows_b.at[q], acc_s.at[dst_b.at[q]],
                                 ssem.at[q], add=True)
                if with_counts:
                    wait_ohgather(q)
                    pltpu.async_copy(oh_b.at[q], cnt_s.at[dst_b.at[q]],
                                     ossem.at[q], add=True)
                wait_scatter(q)
                if with_counts:
                    wait_ohscatter(q)
                stage(j + np.int32(2), q)
            return _I0
        lax.fori_loop(_I0, np.int32(NCH // 2), iter2, _I0)

        for q in (0, 1):
            wait_gather(q)
            if with_counts:
                wait_ohgather(q)

        plsc.subcore_barrier()

        pltpu.sync_copy(
            acc_s.at[pl.ds(rbase, ROWS_PER_TILE)],
            acc_out.at[c, pl.ds(rbase, ROWS_PER_TILE)])
        if with_counts:
            pltpu.sync_copy(
                cnt_s.at[pl.ds(rbase, ROWS_PER_TILE)],
                cnt_out.at[c, pl.ds(rbase, ROWS_PER_TILE)])

    return pl.kernel(
        body,
        out_type=tuple(out_type) if with_counts else out_type[0],
        mesh=mesh,
        compiler_params=pltpu.CompilerParams(
            needs_layout_passes=False, use_tc_tiling_on_sc=False),
        scratch_types=scratch,
    )


_sc_pass_counts = _make_sc_pass(True)
_sc_pass_plain = _make_sc_pass(False)


def _make_dense(with_emb):
    def body(*refs):
        if with_emb:
            x_ref, ew_ref, eb_ref, bs_ref, cf_ref, lp_ref, proj_ref, hl_ref = refs
            h = jnp.dot(x_ref[...], ew_ref[...],
                        preferred_element_type=jnp.float32) + eb_ref[...]
        else:
            x_ref, bs_ref, cf_ref, lp_ref, proj_ref, hl_ref = refs
            h = x_ref[...]
        hl_ref[...] = jnp.dot(h, lp_ref[...], preferred_element_type=jnp.float32)
        for b in range(NB):
            t = jnp.dot(h, bs_ref[b], preferred_element_type=jnp.float32)
            for r in range(R):
                contrib = cf_ref[r, b] * t
                if b == 0:
                    proj_ref[r] = contrib
                else:
                    proj_ref[r] = proj_ref[r] + contrib

    in_specs = [pl.BlockSpec((BN_BLK, D), lambda i: (i, 0))]
    if with_emb:
        in_specs += [
            pl.BlockSpec((D, D), lambda i: (0, 0)),
            pl.BlockSpec((1, D), lambda i: (0, 0)),
        ]
    in_specs += [
        pl.BlockSpec((NB, D, D), lambda i: (0, 0, 0)),
        pl.BlockSpec(memory_space=pltpu.SMEM),
        pl.BlockSpec((D, D), lambda i: (0, 0)),
    ]
    return pl.pallas_call(
        body,
        grid=(GRID,),
        in_specs=in_specs,
        out_specs=(
            pl.BlockSpec((R, BN_BLK, D), lambda i: (0, i, 0)),
            pl.BlockSpec((BN_BLK, D), lambda i: (i, 0)),
        ),
        out_shape=(
            jax.ShapeDtypeStruct((R, N, D), jnp.float32),
            jax.ShapeDtypeStruct((N, D), jnp.float32),
        ),
    )


_dense_emb = _make_dense(True)
_dense_plain = _make_dense(False)


def _bn_relu_block(acc_ref, cnt_ref, hl_ref, g_ref, b_ref):
    p = acc_ref[0] + acc_ref[1]
    cn = cnt_ref[0] + cnt_ref[1]
    norm = jnp.zeros((BN_BLK, 1), jnp.float32)
    for r_ in range(R):
        cr = cn[:, r_:r_ + 1]
        norm = jnp.where(cr > 0, 1.0 / cr, norm)
    o = p * norm + hl_ref[...]
    o = g_ref[...] * (o * INV_BN) + b_ref[...]
    return jnp.maximum(o, 0.0)


def _finalize_body(acc_ref, cnt_ref, hl_ref, g_ref, b_ref, o_ref):
    o_ref[...] = _bn_relu_block(acc_ref, cnt_ref, hl_ref, g_ref, b_ref)


def _fused_body(acc_ref, cnt_ref, hl_ref, g_ref, b_ref, bs_ref, cf_ref,
                lp_ref, proj_ref, hl2_ref):
    h = _bn_relu_block(acc_ref, cnt_ref, hl_ref, g_ref, b_ref)
    hl2_ref[...] = jnp.dot(h, lp_ref[...], preferred_element_type=jnp.float32)
    for b in range(NB):
        t = jnp.dot(h, bs_ref[b], preferred_element_type=jnp.float32)
        for r in range(R):
            contrib = cf_ref[r, b] * t
            if b == 0:
                proj_ref[r] = contrib
            else:
                proj_ref[r] = proj_ref[r] + contrib


_fused_fin_dense = pl.pallas_call(
    _fused_body,
    grid=(GRID,),
    in_specs=[
        pl.BlockSpec((NCORES, BN_BLK, D), lambda i: (0, i, 0)),
        pl.BlockSpec((NCORES, BN_BLK, CW), lambda i: (0, i, 0)),
        pl.BlockSpec((BN_BLK, D), lambda i: (i, 0)),
        pl.BlockSpec((1, D), lambda i: (0, 0)),
        pl.BlockSpec((1, D), lambda i: (0, 0)),
        pl.BlockSpec((NB, D, D), lambda i: (0, 0, 0)),
        pl.BlockSpec(memory_space=pltpu.SMEM),
        pl.BlockSpec((D, D), lambda i: (0, 0)),
    ],
    out_specs=(
        pl.BlockSpec((R, BN_BLK, D), lambda i: (0, i, 0)),
        pl.BlockSpec((BN_BLK, D), lambda i: (i, 0)),
    ),
    out_shape=(
        jax.ShapeDtypeStruct((R, N, D), jnp.float32),
        jax.ShapeDtypeStruct((N, D), jnp.float32),
    ),
)


_finalize = pl.pallas_call(
    _finalize_body,
    grid=(GRID,),
    in_specs=[
        pl.BlockSpec((NCORES, BN_BLK, D), lambda i: (0, i, 0)),
        pl.BlockSpec((NCORES, BN_BLK, CW), lambda i: (0, i, 0)),
        pl.BlockSpec((BN_BLK, D), lambda i: (i, 0)),
        pl.BlockSpec((1, D), lambda i: (0, 0)),
        pl.BlockSpec((1, D), lambda i: (0, 0)),
    ],
    out_specs=pl.BlockSpec((BN_BLK, D), lambda i: (i, 0)),
    out_shape=jax.ShapeDtypeStruct((N, D), jnp.float32),
)


def kernel(x, edge_index, edge_type, emb_w, emb_b, basis0, coef0, loop0,
           gamma0, beta0, basis1, coef1, loop1, gamma1, beta1):
    with jax.enable_x64(False):
        return _kernel_32(x, edge_index, edge_type, emb_w, emb_b, basis0,
                          coef0, loop0, gamma0, beta0, basis1, coef1, loop1,
                          gamma1, beta1)


def _kernel_32(x, edge_index, edge_type, emb_w, emb_b, basis0, coef0, loop0,
               gamma0, beta0, basis1, coef1, loop1, gamma1, beta1):
    src = edge_index[0].astype(jnp.int32)
    dst = edge_index[1].astype(jnp.int32)
    typ = edge_type.astype(jnp.int32)

    npad = E_PAD - E
    ar = jnp.arange(npad, dtype=jnp.int32)
    src_p = jnp.concatenate([src, ar % N]).reshape(E_PAD // CK, CK)
    dst_p = jnp.concatenate([dst, N + ar % (N_ACC - N)]).reshape(E_PAD // CK, CK)
    typ_p = jnp.concatenate([typ, jnp.zeros((npad,), jnp.int32)]).reshape(
        E_PAD // CK, CK)
    edges = jnp.concatenate(
        [jnp.stack([src_p, dst_p, typ_p], axis=1),
         jnp.zeros((2, 3, CK), jnp.int32)], axis=0)

    emb_b2 = emb_b.reshape(1, D)
    g0 = gamma0.reshape(1, D)
    b0 = beta0.reshape(1, D)
    g1 = gamma1.reshape(1, D)
    b1 = beta1.reshape(1, D)

    proj1, hl1 = _dense_emb(x, emb_w, emb_b2, basis0, coef0, loop0)
    acc1, cnt = _sc_pass_counts(proj1.reshape(R * N, D), edges,
                                jnp.asarray(_OHTAB))
    proj2, hl2 = _fused_fin_dense(acc1, cnt, hl1, g0, b0, basis1, coef1, loop1)
    acc2 = _sc_pass_plain(proj2.reshape(R * N, D), edges)
    h2 = _finalize(acc2, cnt, hl2, g1, b1)
    return h2


# speedup vs baseline: 45.5561x; 1.1022x over previous
import math

import numpy as np

import jax
import jax.numpy as jnp
from jax import lax
from jax.experimental import pallas as pl
from jax.experimental.pallas import tpu as pltpu
from jax.experimental.pallas import tpu_sc as plsc

N = 10000
D = 128
R = 8
NB = 4
E = 320000
EPS = 1e-3
INV_BN = float(1.0 / math.sqrt(1.0 + EPS))

NCORES = 2
NSUB = 16
NTILES = NCORES * NSUB
CK = 128
NCH = 80
ET = NCH * CK
E_PAD = NTILES * ET
N_ACC = 10240
ROWS_PER_TILE = N_ACC // NSUB
CW = 16
BN_BLK = 1000
GRID = N // BN_BLK

_I0 = np.int32(0)
_ICK = np.int32(CK)
_ICH = np.int32(NCH)
_IN = np.int32(N)
_INSUB = np.int32(NSUB)
_IRPT = np.int32(ROWS_PER_TILE)
_I32F = np.int32(32)
_I31 = np.int32(31)

_OHTAB = np.repeat(np.eye(CW, dtype=np.float32)[:R], 32, axis=0)


def _make_sc_pass(with_counts):
    mesh = plsc.VectorSubcoreMesh(core_axis_name="c", subcore_axis_name="s")
    out_type = [jax.ShapeDtypeStruct((NCORES, N_ACC, D), jnp.float32)]
    if with_counts:
        out_type.append(jax.ShapeDtypeStruct((NCORES, N_ACC, CW), jnp.float32))
    scratch = [
        pltpu.VMEM((2, 1, 3, CK), jnp.int32),
        pltpu.VMEM((2, CK), jnp.int32),
        pltpu.VMEM((2, CK), jnp.int32),
        pltpu.VMEM((2, CK, D), jnp.float32),
        pltpu.VMEM_SHARED((N_ACC, D), jnp.float32),
        pltpu.SemaphoreType.DMA((2,)),
        pltpu.SemaphoreType.DMA((2,)),
        pltpu.SemaphoreType.DMA((2,)),
    ]
    if with_counts:
        scratch += [
            pltpu.VMEM((2, CK), jnp.int32),
            pltpu.VMEM((2, CK, CW), jnp.float32),
            pltpu.VMEM_SHARED((N_ACC, CW), jnp.float32),
            pltpu.VMEM_SHARED((R * 32, CW), jnp.float32),
            pltpu.SemaphoreType.DMA((2,)),
            pltpu.SemaphoreType.DMA((2,)),
        ]

    def body(proj_hbm, edges_hbm, *rest):
        if with_counts:
            (ohtab_hbm, acc_out, cnt_out, ebuf, dst_b, key_b, rows_b, acc_s,
             gsem, ssem, esem, key2_b, oh_b, cnt_s, ohtab_s, ogsem,
             ossem) = rest
        else:
            acc_out, ebuf, dst_b, key_b, rows_b, acc_s, gsem, ssem, esem = rest
        c = lax.axis_index("c")
        s = lax.axis_index("s")
        wid = c * _INSUB + s
        rbase = s * _IRPT

        def zrow(i, _):
            for v in range(D // 16):
                rows_b[0, i, pl.ds(v * 16, 16)] = jnp.zeros((16,), jnp.float32)
            return _I0
        lax.fori_loop(_I0, _ICK, zrow, _I0)
        for blk in range(ROWS_PER_TILE // CK):
            pltpu.sync_copy(rows_b.at[0],
                            acc_s.at[pl.ds(rbase + np.int32(blk * CK), CK)])
        if with_counts:
            def zoh(i, _):
                oh_b[0, i, pl.ds(0, CW)] = jnp.zeros((CW,), jnp.float32)
                return _I0
            lax.fori_loop(_I0, _ICK, zoh, _I0)
            for blk in range(ROWS_PER_TILE // CK):
                pltpu.sync_copy(
                    oh_b.at[0], cnt_s.at[pl.ds(rbase + np.int32(blk * CK), CK)])
            pltpu.sync_copy(ohtab_hbm, ohtab_s)

        plsc.subcore_barrier()

        def issue_estage(j, q):
            row = wid * _ICH + j
            pltpu.async_copy(edges_hbm.at[pl.ds(row, 1)], ebuf.at[q],
                             esem.at[q])

        def stage(j, q):
            pltpu.make_async_copy(edges_hbm.at[pl.ds(_I0, 1)], ebuf.at[q],
                                  esem.at[q]).wait()
            for v in range(CK // 16):
                sv = ebuf[q, 0, 0, pl.ds(v * 16, 16)]
                dv = ebuf[q, 0, 1, pl.ds(v * 16, 16)]
                tv = ebuf[q, 0, 2, pl.ds(v * 16, 16)]
                key_b[q, pl.ds(v * 16, 16)] = tv * _IN + sv
                dst_b[q, pl.ds(v * 16, 16)] = dv
                if with_counts:
                    key2_b[q, pl.ds(v * 16, 16)] = tv * _I32F + (sv & _I31)
            issue_estage(j + np.int32(2), q)
            pltpu.async_copy(proj_hbm.at[key_b.at[q]], rows_b.at[q], gsem.at[q])
            if with_counts:
                pltpu.async_copy(ohtab_s.at[key2_b.at[q]], oh_b.at[q],
                                 ogsem.at[q])

        def wait_gather(q):
            pltpu.make_async_copy(proj_hbm.at[key_b.at[q]], rows_b.at[q],
                                  gsem.at[q]).wait()

        def wait_scatter(q):
            pltpu.make_async_copy(rows_b.at[q], acc_s.at[dst_b.at[q]],
                                  ssem.at[q]).wait()

        def wait_ohgather(q):
            pltpu.make_async_copy(ohtab_s.at[key2_b.at[q]], oh_b.at[q],
                                  ogsem.at[q]).wait()

        def wait_ohscatter(q):
            pltpu.make_async_copy(oh_b.at[q], cnt_s.at[dst_b.at[q]],
                                  ossem.at[q]).wait()

        issue_estage(_I0, 0)
        issue_estage(np.int32(1), 1)
        stage(_I0, 0)
        stage(np.int32(1), 1)

        def iter2(g, _):
            for q in (0, 1):
                j = g * np.int32(2) + np.int32(q)
                wait_gather(q)
                pltpu.async_copy(rows_b.at[q], acc_s.at[dst_b.at[q]],
                                 ssem.at[q], add=True)
                if with_counts:
                    wait_ohgather(q)
                    pltpu.async_copy(oh_b.at[q], cnt_s.at[dst_b.at[q]],
                                     ossem.at[q], add=True)
                wait_scatter(q)
                if with_counts:
                    wait_ohscatter(q)
                stage(j + np.int32(2), q)
            return _I0
        lax.fori_loop(_I0, np.int32(NCH // 2), iter2, _I0)

        for q in (0, 1):
            wait_gather(q)
            pltpu.make_async_copy(edges_hbm.at[pl.ds(_I0, 1)], ebuf.at[q],
                                  esem.at[q]).wait()
            if with_counts:
                wait_ohgather(q)

        plsc.subcore_barrier()

        pltpu.sync_copy(
            acc_s.at[pl.ds(rbase, ROWS_PER_TILE)],
            acc_out.at[c, pl.ds(rbase, ROWS_PER_TILE)])
        if with_counts:
            pltpu.sync_copy(
                cnt_s.at[pl.ds(rbase, ROWS_PER_TILE)],
                cnt_out.at[c, pl.ds(rbase, ROWS_PER_TILE)])

    return pl.kernel(
        body,
        out_type=tuple(out_type) if with_counts else out_type[0],
        mesh=mesh,
        compiler_params=pltpu.CompilerParams(
            needs_layout_passes=False, use_tc_tiling_on_sc=False),
        scratch_types=scratch,
    )


_sc_pass_counts = _make_sc_pass(True)
_sc_pass_plain = _make_sc_pass(False)


def _make_dense(with_emb):
    def body(*refs):
        if with_emb:
            x_ref, ew_ref, eb_ref, bs_ref, cf_ref, lp_ref, proj_ref, hl_ref = refs
            h = jnp.dot(x_ref[...], ew_ref[...],
                        preferred_element_type=jnp.float32) + eb_ref[...]
        else:
            x_ref, bs_ref, cf_ref, lp_ref, proj_ref, hl_ref = refs
            h = x_ref[...]
        hl_ref[...] = jnp.dot(h, lp_ref[...], preferred_element_type=jnp.float32)
        for b in range(NB):
            t = jnp.dot(h, bs_ref[b], preferred_element_type=jnp.float32)
            for r in range(R):
                contrib = cf_ref[r, b] * t
                if b == 0:
                    proj_ref[r] = contrib
                else:
                    proj_ref[r] = proj_ref[r] + contrib

    in_specs = [pl.BlockSpec((BN_BLK, D), lambda i: (i, 0))]
    if with_emb:
        in_specs += [
            pl.BlockSpec((D, D), lambda i: (0, 0)),
            pl.BlockSpec((1, D), lambda i: (0, 0)),
        ]
    in_specs += [
        pl.BlockSpec((NB, D, D), lambda i: (0, 0, 0)),
        pl.BlockSpec(memory_space=pltpu.SMEM),
        pl.BlockSpec((D, D), lambda i: (0, 0)),
    ]
    return pl.pallas_call(
        body,
        grid=(GRID,),
        in_specs=in_specs,
        out_specs=(
            pl.BlockSpec((R, BN_BLK, D), lambda i: (0, i, 0)),
            pl.BlockSpec((BN_BLK, D), lambda i: (i, 0)),
        ),
        out_shape=(
            jax.ShapeDtypeStruct((R, N, D), jnp.float32),
            jax.ShapeDtypeStruct((N, D), jnp.float32),
        ),
    )


_dense_emb = _make_dense(True)
_dense_plain = _make_dense(False)


def _bn_relu_block(acc_ref, cnt_ref, hl_ref, g_ref, b_ref):
    p = acc_ref[0] + acc_ref[1]
    cn = cnt_ref[0] + cnt_ref[1]
    norm = jnp.zeros((BN_BLK, 1), jnp.float32)
    for r_ in range(R):
        cr = cn[:, r_:r_ + 1]
        norm = jnp.where(cr > 0, 1.0 / cr, norm)
    o = p * norm + hl_ref[...]
    o = g_ref[...] * (o * INV_BN) + b_ref[...]
    return jnp.maximum(o, 0.0)


def _finalize_body(acc_ref, cnt_ref, hl_ref, g_ref, b_ref, o_ref):
    o_ref[...] = _bn_relu_block(acc_ref, cnt_ref, hl_ref, g_ref, b_ref)


def _fused_body(acc_ref, cnt_ref, hl_ref, g_ref, b_ref, bs_ref, cf_ref,
                lp_ref, proj_ref, hl2_ref):
    h = _bn_relu_block(acc_ref, cnt_ref, hl_ref, g_ref, b_ref)
    hl2_ref[...] = jnp.dot(h, lp_ref[...], preferred_element_type=jnp.float32)
    for b in range(NB):
        t = jnp.dot(h, bs_ref[b], preferred_element_type=jnp.float32)
        for r in range(R):
            contrib = cf_ref[r, b] * t
            if b == 0:
                proj_ref[r] = contrib
            else:
                proj_ref[r] = proj_ref[r] + contrib


_fused_fin_dense = pl.pallas_call(
    _fused_body,
    grid=(GRID,),
    in_specs=[
        pl.BlockSpec((NCORES, BN_BLK, D), lambda i: (0, i, 0)),
        pl.BlockSpec((NCORES, BN_BLK, CW), lambda i: (0, i, 0)),
        pl.BlockSpec((BN_BLK, D), lambda i: (i, 0)),
        pl.BlockSpec((1, D), lambda i: (0, 0)),
        pl.BlockSpec((1, D), lambda i: (0, 0)),
        pl.BlockSpec((NB, D, D), lambda i: (0, 0, 0)),
        pl.BlockSpec(memory_space=pltpu.SMEM),
        pl.BlockSpec((D, D), lambda i: (0, 0)),
    ],
    out_specs=(
        pl.BlockSpec((R, BN_BLK, D), lambda i: (0, i, 0)),
        pl.BlockSpec((BN_BLK, D), lambda i: (i, 0)),
    ),
    out_shape=(
        jax.ShapeDtypeStruct((R, N, D), jnp.float32),
        jax.ShapeDtypeStruct((N, D), jnp.float32),
    ),
)


_finalize = pl.pallas_call(
    _finalize_body,
    grid=(GRID,),
    in_specs=[
        pl.BlockSpec((NCORES, BN_BLK, D), lambda i: (0, i, 0)),
        pl.BlockSpec((NCORES, BN_BLK, CW), lambda i: (0, i, 0)),
        pl.BlockSpec((BN_BLK, D), lambda i: (i, 0)),
        pl.BlockSpec((1, D), lambda i: (0, 0)),
        pl.BlockSpec((1, D), lambda i: (0, 0)),
    ],
    out_specs=pl.BlockSpec((BN_BLK, D), lambda i: (i, 0)),
    out_shape=jax.ShapeDtypeStruct((N, D), jnp.float32),
)


def kernel(x, edge_index, edge_type, emb_w, emb_b, basis0, coef0, loop0,
           gamma0, beta0, basis1, coef1, loop1, gamma1, beta1):
    with jax.enable_x64(False):
        return _kernel_32(x, edge_index, edge_type, emb_w, emb_b, basis0,
                          coef0, loop0, gamma0, beta0, basis1, coef1, loop1,
                          gamma1, beta1)


def _kernel_32(x, edge_index, edge_type, emb_w, emb_b, basis0, coef0, loop0,
               gamma0, beta0, basis1, coef1, loop1, gamma1, beta1):
    src = edge_index[0].astype(jnp.int32)
    dst = edge_index[1].astype(jnp.int32)
    typ = edge_type.astype(jnp.int32)

    npad = E_PAD - E
    ar = jnp.arange(npad, dtype=jnp.int32)
    src_p = jnp.concatenate([src, ar % N]).reshape(E_PAD // CK, CK)
    dst_p = jnp.concatenate([dst, N + ar % (N_ACC - N)]).reshape(E_PAD // CK, CK)
    typ_p = jnp.concatenate([typ, jnp.zeros((npad,), jnp.int32)]).reshape(
        E_PAD // CK, CK)
    edges = jnp.concatenate(
        [jnp.stack([src_p, dst_p, typ_p], axis=1),
         jnp.zeros((4, 3, CK), jnp.int32)], axis=0)

    emb_b2 = emb_b.reshape(1, D)
    g0 = gamma0.reshape(1, D)
    b0 = beta0.reshape(1, D)
    g1 = gamma1.reshape(1, D)
    b1 = beta1.reshape(1, D)

    proj1, hl1 = _dense_emb(x, emb_w, emb_b2, basis0, coef0, loop0)
    acc1, cnt = _sc_pass_counts(proj1.reshape(R * N, D), edges,
                                jnp.asarray(_OHTAB))
    proj2, hl2 = _fused_fin_dense(acc1, cnt, hl1, g0, b0, basis1, coef1, loop1)
    acc2 = _sc_pass_plain(proj2.reshape(R * N, D), edges)
    h2 = _finalize(acc2, cnt, hl2, g1, b1)
    return h2


# speedup vs baseline: 45.7097x; 1.0034x over previous
import math

import numpy as np

import jax
import jax.numpy as jnp
from jax import lax
from jax.experimental import pallas as pl
from jax.experimental.pallas import tpu as pltpu
from jax.experimental.pallas import tpu_sc as plsc

N = 10000
D = 128
R = 8
NB = 4
E = 320000
EPS = 1e-3
INV_BN = float(1.0 / math.sqrt(1.0 + EPS))

NCORES = 2
NSUB = 16
NTILES = NCORES * NSUB
CK = 128
NCH = 80
ET = NCH * CK
E_PAD = NTILES * ET
N_ACC = 10240
ROWS_PER_TILE = N_ACC // NSUB
CW = 16
BN_BLK = 1000
GRID = N // BN_BLK

_I0 = np.int32(0)
_ICK = np.int32(CK)
_ICH = np.int32(NCH)
_IN = np.int32(N)
_INSUB = np.int32(NSUB)
_IRPT = np.int32(ROWS_PER_TILE)
_I32F = np.int32(32)
_I31 = np.int32(31)

_OHTAB = np.repeat(np.eye(CW, dtype=np.float32)[:R], 32, axis=0)


def _make_sc_pass(with_counts):
    mesh = plsc.VectorSubcoreMesh(core_axis_name="c", subcore_axis_name="s")
    out_type = [jax.ShapeDtypeStruct((NCORES, N_ACC, D), jnp.float32)]
    if with_counts:
        out_type.append(jax.ShapeDtypeStruct((NCORES, N_ACC, CW), jnp.float32))
    scratch = [
        pltpu.VMEM((2, 1, 3, CK), jnp.int32),
        pltpu.VMEM((2, 2, CK), jnp.int32),
        pltpu.VMEM((2, CK), jnp.int32),
        pltpu.VMEM((2, CK, D), jnp.float32),
        pltpu.VMEM_SHARED((N_ACC, D), jnp.float32),
        pltpu.SemaphoreType.DMA((2,)),
        pltpu.SemaphoreType.DMA((2,)),
        pltpu.SemaphoreType.DMA((2,)),
    ]
    if with_counts:
        scratch += [
            pltpu.VMEM((2, CK), jnp.int32),
            pltpu.VMEM((2, CK, CW), jnp.float32),
            pltpu.VMEM_SHARED((N_ACC, CW), jnp.float32),
            pltpu.VMEM_SHARED((R * 32, CW), jnp.float32),
            pltpu.SemaphoreType.DMA((2,)),
            pltpu.SemaphoreType.DMA((2,)),
        ]

    def body(proj_hbm, edges_hbm, *rest):
        if with_counts:
            (ohtab_hbm, acc_out, cnt_out, ebuf, dst_b, key_b, rows_b, acc_s,
             gsem, ssem, esem, key2_b, oh_b, cnt_s, ohtab_s, ogsem,
             ossem) = rest
        else:
            acc_out, ebuf, dst_b, key_b, rows_b, acc_s, gsem, ssem, esem = rest
        c = lax.axis_index("c")
        s = lax.axis_index("s")
        wid = c * _INSUB + s
        rbase = s * _IRPT

        def zrow(i, _):
            for v in range(D // 16):
                rows_b[0, i, pl.ds(v * 16, 16)] = jnp.zeros((16,), jnp.float32)
            return _I0
        lax.fori_loop(_I0, _ICK, zrow, _I0)
        for blk in range(ROWS_PER_TILE // CK):
            pltpu.sync_copy(rows_b.at[0],
                            acc_s.at[pl.ds(rbase + np.int32(blk * CK), CK)])
        if with_counts:
            def zoh(i, _):
                oh_b[0, i, pl.ds(0, CW)] = jnp.zeros((CW,), jnp.float32)
                return _I0
            lax.fori_loop(_I0, _ICK, zoh, _I0)
            for blk in range(ROWS_PER_TILE // CK):
                pltpu.sync_copy(
                    oh_b.at[0], cnt_s.at[pl.ds(rbase + np.int32(blk * CK), CK)])
            pltpu.sync_copy(ohtab_hbm, ohtab_s)

        plsc.subcore_barrier()

        def issue_estage(j, q):
            row = wid * _ICH + j
            pltpu.async_copy(edges_hbm.at[pl.ds(row, 1)], ebuf.at[q],
                             esem.at[q])

        def stage_compute(j, q, ph):
            pltpu.make_async_copy(edges_hbm.at[pl.ds(_I0, 1)], ebuf.at[q],
                                  esem.at[q]).wait()
            for v in range(CK // 16):
                sv = ebuf[q, 0, 0, pl.ds(v * 16, 16)]
                dv = ebuf[q, 0, 1, pl.ds(v * 16, 16)]
                tv = ebuf[q, 0, 2, pl.ds(v * 16, 16)]
                key_b[q, pl.ds(v * 16, 16)] = tv * _IN + sv
                dst_b[q, ph, pl.ds(v * 16, 16)] = dv
                if with_counts:
                    key2_b[q, pl.ds(v * 16, 16)] = tv * _I32F + (sv & _I31)
            issue_estage(j + np.int32(2), q)

        def issue_gathers(q):
            pltpu.async_copy(proj_hbm.at[key_b.at[q]], rows_b.at[q], gsem.at[q])
            if with_counts:
                pltpu.async_copy(ohtab_s.at[key2_b.at[q]], oh_b.at[q],
                                 ogsem.at[q])

        def wait_gather(q):
            pltpu.make_async_copy(proj_hbm.at[key_b.at[q]], rows_b.at[q],
                                  gsem.at[q]).wait()

        def wait_scatter(q, ph):
            pltpu.make_async_copy(rows_b.at[q], acc_s.at[dst_b.at[q, ph]],
                                  ssem.at[q]).wait()

        def wait_ohgather(q):
            pltpu.make_async_copy(ohtab_s.at[key2_b.at[q]], oh_b.at[q],
                                  ogsem.at[q]).wait()

        def wait_ohscatter(q, ph):
            pltpu.make_async_copy(oh_b.at[q], cnt_s.at[dst_b.at[q, ph]],
                                  ossem.at[q]).wait()

        issue_estage(_I0, 0)
        issue_estage(np.int32(1), 1)
        stage_compute(_I0, 0, 0)
        issue_gathers(0)
        stage_compute(np.int32(1), 1, 0)
        issue_gathers(1)

        def iter4(m, _):
            for ph in (0, 1):
                for q in (0, 1):
                    j = m * np.int32(4) + np.int32(2 * ph + q)
                    wait_gather(q)
                    pltpu.async_copy(rows_b.at[q], acc_s.at[dst_b.at[q, ph]],
                                     ssem.at[q], add=True)
                    if with_counts:
                        wait_ohgather(q)
                        pltpu.async_copy(oh_b.at[q],
                                         cnt_s.at[dst_b.at[q, ph]],
                                         ossem.at[q], add=True)
                    stage_compute(j + np.int32(2), q, ph ^ 1)
                    wait_scatter(q, ph)
                    if with_counts:
                        wait_ohscatter(q, ph)
                    issue_gathers(q)
            return _I0
        lax.fori_loop(_I0, np.int32(NCH // 4), iter4, _I0)

        for q in (0, 1):
            wait_gather(q)
            pltpu.make_async_copy(edges_hbm.at[pl.ds(_I0, 1)], ebuf.at[q],
                                  esem.at[q]).wait()
            if with_counts:
                wait_ohgather(q)

        plsc.subcore_barrier()

        pltpu.sync_copy(
            acc_s.at[pl.ds(rbase, ROWS_PER_TILE)],
            acc_out.at[c, pl.ds(rbase, ROWS_PER_TILE)])
        if with_counts:
            pltpu.sync_copy(
                cnt_s.at[pl.ds(rbase, ROWS_PER_TILE)],
                cnt_out.at[c, pl.ds(rbase, ROWS_PER_TILE)])

    return pl.kernel(
        body,
        out_type=tuple(out_type) if with_counts else out_type[0],
        mesh=mesh,
        compiler_params=pltpu.CompilerParams(
            needs_layout_passes=False, use_tc_tiling_on_sc=False),
        scratch_types=scratch,
    )


_sc_pass_counts = _make_sc_pass(True)
_sc_pass_plain = _make_sc_pass(False)


def _make_dense(with_emb):
    def body(*refs):
        if with_emb:
            x_ref, ew_ref, eb_ref, bs_ref, cf_ref, lp_ref, proj_ref, hl_ref = refs
            h = jnp.dot(x_ref[...], ew_ref[...],
                        preferred_element_type=jnp.float32) + eb_ref[...]
        else:
            x_ref, bs_ref, cf_ref, lp_ref, proj_ref, hl_ref = refs
            h = x_ref[...]
        hl_ref[...] = jnp.dot(h, lp_ref[...], preferred_element_type=jnp.float32)
        for b in range(NB):
            t = jnp.dot(h, bs_ref[b], preferred_element_type=jnp.float32)
            for r in range(R):
                contrib = cf_ref[r, b] * t
                if b == 0:
                    proj_ref[r] = contrib
                else:
                    proj_ref[r] = proj_ref[r] + contrib

    in_specs = [pl.BlockSpec((BN_BLK, D), lambda i: (i, 0))]
    if with_emb:
        in_specs += [
            pl.BlockSpec((D, D), lambda i: (0, 0)),
            pl.BlockSpec((1, D), lambda i: (0, 0)),
        ]
    in_specs += [
        pl.BlockSpec((NB, D, D), lambda i: (0, 0, 0)),
        pl.BlockSpec(memory_space=pltpu.SMEM),
        pl.BlockSpec((D, D), lambda i: (0, 0)),
    ]
    return pl.pallas_call(
        body,
        grid=(GRID,),
        in_specs=in_specs,
        out_specs=(
            pl.BlockSpec((R, BN_BLK, D), lambda i: (0, i, 0)),
            pl.BlockSpec((BN_BLK, D), lambda i: (i, 0)),
        ),
        out_shape=(
            jax.ShapeDtypeStruct((R, N, D), jnp.float32),
            jax.ShapeDtypeStruct((N, D), jnp.float32),
        ),
    )


_dense_emb = _make_dense(True)
_dense_plain = _make_dense(False)


def _bn_relu_block(acc_ref, cnt_ref, hl_ref, g_ref, b_ref):
    p = acc_ref[0] + acc_ref[1]
    cn = cnt_ref[0] + cnt_ref[1]
    norm = jnp.zeros((BN_BLK, 1), jnp.float32)
    for r_ in range(R):
        cr = cn[:, r_:r_ + 1]
        norm = jnp.where(cr > 0, 1.0 / cr, norm)
    o = p * norm + hl_ref[...]
    o = g_ref[...] * (o * INV_BN) + b_ref[...]
    return jnp.maximum(o, 0.0)


def _finalize_body(acc_ref, cnt_ref, hl_ref, g_ref, b_ref, o_ref):
    o_ref[...] = _bn_relu_block(acc_ref, cnt_ref, hl_ref, g_ref, b_ref)


def _fused_body(acc_ref, cnt_ref, hl_ref, g_ref, b_ref, bs_ref, cf_ref,
                lp_ref, proj_ref, hl2_ref):
    h = _bn_relu_block(acc_ref, cnt_ref, hl_ref, g_ref, b_ref)
    hl2_ref[...] = jnp.dot(h, lp_ref[...], preferred_element_type=jnp.float32)
    for b in range(NB):
        t = jnp.dot(h, bs_ref[b], preferred_element_type=jnp.float32)
        for r in range(R):
            contrib = cf_ref[r, b] * t
            if b == 0:
                proj_ref[r] = contrib
            else:
                proj_ref[r] = proj_ref[r] + contrib


_fused_fin_dense = pl.pallas_call(
    _fused_body,
    grid=(GRID,),
    in_specs=[
        pl.BlockSpec((NCORES, BN_BLK, D), lambda i: (0, i, 0)),
        pl.BlockSpec((NCORES, BN_BLK, CW), lambda i: (0, i, 0)),
        pl.BlockSpec((BN_BLK, D), lambda i: (i, 0)),
        pl.BlockSpec((1, D), lambda i: (0, 0)),
        pl.BlockSpec((1, D), lambda i: (0, 0)),
        pl.BlockSpec((NB, D, D), lambda i: (0, 0, 0)),
        pl.BlockSpec(memory_space=pltpu.SMEM),
        pl.BlockSpec((D, D), lambda i: (0, 0)),
    ],
    out_specs=(
        pl.BlockSpec((R, BN_BLK, D), lambda i: (0, i, 0)),
        pl.BlockSpec((BN_BLK, D), lambda i: (i, 0)),
    ),
    out_shape=(
        jax.ShapeDtypeStruct((R, N, D), jnp.float32),
        jax.ShapeDtypeStruct((N, D), jnp.float32),
    ),
)


_finalize = pl.pallas_call(
    _finalize_body,
    grid=(GRID,),
    in_specs=[
        pl.BlockSpec((NCORES, BN_BLK, D), lambda i: (0, i, 0)),
        pl.BlockSpec((NCORES, BN_BLK, CW), lambda i: (0, i, 0)),
        pl.BlockSpec((BN_BLK, D), lambda i: (i, 0)),
        pl.BlockSpec((1, D), lambda i: (0, 0)),
        pl.BlockSpec((1, D), lambda i: (0, 0)),
    ],
    out_specs=pl.BlockSpec((BN_BLK, D), lambda i: (i, 0)),
    out_shape=jax.ShapeDtypeStruct((N, D), jnp.float32),
)


def kernel(x, edge_index, edge_type, emb_w, emb_b, basis0, coef0, loop0,
           gamma0, beta0, basis1, coef1, loop1, gamma1, beta1):
    with jax.enable_x64(False):
        return _kernel_32(x, edge_index, edge_type, emb_w, emb_b, basis0,
                          coef0, loop0, gamma0, beta0, basis1, coef1, loop1,
                          gamma1, beta1)


def _kernel_32(x, edge_index, edge_type, emb_w, emb_b, basis0, coef0, loop0,
               gamma0, beta0, basis1, coef1, loop1, gamma1, beta1):
    src = edge_index[0].astype(jnp.int32)
    dst = edge_index[1].astype(jnp.int32)
    typ = edge_type.astype(jnp.int32)

    npad = E_PAD - E
    ar = jnp.arange(npad, dtype=jnp.int32)
    src_p = jnp.concatenate([src, ar % N]).reshape(E_PAD // CK, CK)
    dst_p = jnp.concatenate([dst, N + ar % (N_ACC - N)]).reshape(E_PAD // CK, CK)
    typ_p = jnp.concatenate([typ, jnp.zeros((npad,), jnp.int32)]).reshape(
        E_PAD // CK, CK)
    edges = jnp.concatenate(
        [jnp.stack([src_p, dst_p, typ_p], axis=1),
         jnp.zeros((4, 3, CK), jnp.int32)], axis=0)

    emb_b2 = emb_b.reshape(1, D)
    g0 = gamma0.reshape(1, D)
    b0 = beta0.reshape(1, D)
    g1 = gamma1.reshape(1, D)
    b1 = beta1.reshape(1, D)

    proj1, hl1 = _dense_emb(x, emb_w, emb_b2, basis0, coef0, loop0)
    acc1, cnt = _sc_pass_counts(proj1.reshape(R * N, D), edges,
                                jnp.asarray(_OHTAB))
    proj2, hl2 = _fused_fin_dense(acc1, cnt, hl1, g0, b0, basis1, coef1, loop1)
    acc2 = _sc_pass_plain(proj2.reshape(R * N, D), edges)
    h2 = _finalize(acc2, cnt, hl2, g1, b1)
    return h2


# speedup vs baseline: 46.2004x; 1.0107x over previous
import math

import numpy as np

import jax
import jax.numpy as jnp
from jax import lax
from jax.experimental import pallas as pl
from jax.experimental.pallas import tpu as pltpu
from jax.experimental.pallas import tpu_sc as plsc

N = 10000
D = 128
R = 8
NB = 4
E = 320000
EPS = 1e-3
INV_BN = float(1.0 / math.sqrt(1.0 + EPS))

NCORES = 2
NSUB = 16
NTILES = NCORES * NSUB
CK = 128
NCH = 80
ET = NCH * CK
E_PAD = NTILES * ET
N_ACC = 10240
ROWS_PER_TILE = N_ACC // NSUB
CW = 16
BN_BLK = 2000
GRID = N // BN_BLK

_I0 = np.int32(0)
_ICK = np.int32(CK)
_ICH = np.int32(NCH)
_IN = np.int32(N)
_INSUB = np.int32(NSUB)
_IRPT = np.int32(ROWS_PER_TILE)
_I32F = np.int32(32)
_I31 = np.int32(31)

_OHTAB = np.repeat(np.eye(CW, dtype=np.float32)[:R], 32, axis=0)


def _make_sc_pass(with_counts):
    mesh = plsc.VectorSubcoreMesh(core_axis_name="c", subcore_axis_name="s")
    out_type = [jax.ShapeDtypeStruct((NCORES, N_ACC, D), jnp.float32)]
    if with_counts:
        out_type.append(jax.ShapeDtypeStruct((NCORES, N_ACC, CW), jnp.float32))
    scratch = [
        pltpu.VMEM((2, 1, 3, CK), jnp.int32),
        pltpu.VMEM((2, 2, CK), jnp.int32),
        pltpu.VMEM((2, CK), jnp.int32),
        pltpu.VMEM((2, CK, D), jnp.float32),
        pltpu.VMEM_SHARED((N_ACC, D), jnp.float32),
        pltpu.SemaphoreType.DMA((2,)),
        pltpu.SemaphoreType.DMA((2,)),
        pltpu.SemaphoreType.DMA((2,)),
    ]
    if with_counts:
        scratch += [
            pltpu.VMEM((2, CK), jnp.int32),
            pltpu.VMEM((2, CK, CW), jnp.float32),
            pltpu.VMEM_SHARED((N_ACC, CW), jnp.float32),
            pltpu.VMEM_SHARED((R * 32, CW), jnp.float32),
            pltpu.SemaphoreType.DMA((2,)),
            pltpu.SemaphoreType.DMA((2,)),
        ]

    def body(proj_hbm, edges_hbm, *rest):
        if with_counts:
            (ohtab_hbm, acc_out, cnt_out, ebuf, dst_b, key_b, rows_b, acc_s,
             gsem, ssem, esem, key2_b, oh_b, cnt_s, ohtab_s, ogsem,
             ossem) = rest
        else:
            acc_out, ebuf, dst_b, key_b, rows_b, acc_s, gsem, ssem, esem = rest
        c = lax.axis_index("c")
        s = lax.axis_index("s")
        wid = c * _INSUB + s
        rbase = s * _IRPT

        def zrow(i, _):
            for v in range(D // 16):
                rows_b[0, i, pl.ds(v * 16, 16)] = jnp.zeros((16,), jnp.float32)
            return _I0
        lax.fori_loop(_I0, _ICK, zrow, _I0)
        for blk in range(ROWS_PER_TILE // CK):
            pltpu.sync_copy(rows_b.at[0],
                            acc_s.at[pl.ds(rbase + np.int32(blk * CK), CK)])
        if with_counts:
            def zoh(i, _):
                oh_b[0, i, pl.ds(0, CW)] = jnp.zeros((CW,), jnp.float32)
                return _I0
            lax.fori_loop(_I0, _ICK, zoh, _I0)
            for blk in range(ROWS_PER_TILE // CK):
                pltpu.sync_copy(
                    oh_b.at[0], cnt_s.at[pl.ds(rbase + np.int32(blk * CK), CK)])
            pltpu.sync_copy(ohtab_hbm, ohtab_s)

        plsc.subcore_barrier()

        def issue_estage(j, q):
            row = wid * _ICH + j
            pltpu.async_copy(edges_hbm.at[pl.ds(row, 1)], ebuf.at[q],
                             esem.at[q])

        def stage_compute(j, q, ph):
            pltpu.make_async_copy(edges_hbm.at[pl.ds(_I0, 1)], ebuf.at[q],
                                  esem.at[q]).wait()
            for v in range(CK // 16):
                sv = ebuf[q, 0, 0, pl.ds(v * 16, 16)]
                dv = ebuf[q, 0, 1, pl.ds(v * 16, 16)]
                tv = ebuf[q, 0, 2, pl.ds(v * 16, 16)]
                key_b[q, pl.ds(v * 16, 16)] = tv * _IN + sv
                dst_b[q, ph, pl.ds(v * 16, 16)] = dv
                if with_counts:
                    key2_b[q, pl.ds(v * 16, 16)] = tv * _I32F + (sv & _I31)
            issue_estage(j + np.int32(2), q)

        def issue_gathers(q):
            pltpu.async_copy(proj_hbm.at[key_b.at[q]], rows_b.at[q], gsem.at[q])
            if with_counts:
                pltpu.async_copy(ohtab_s.at[key2_b.at[q]], oh_b.at[q],
                                 ogsem.at[q])

        def wait_gather(q):
            pltpu.make_async_copy(proj_hbm.at[key_b.at[q]], rows_b.at[q],
                                  gsem.at[q]).wait()

        def wait_scatter(q, ph):
            pltpu.make_async_copy(rows_b.at[q], acc_s.at[dst_b.at[q, ph]],
                                  ssem.at[q]).wait()

        def wait_ohgather(q):
            pltpu.make_async_copy(ohtab_s.at[key2_b.at[q]], oh_b.at[q],
                                  ogsem.at[q]).wait()

        def wait_ohscatter(q, ph):
            pltpu.make_async_copy(oh_b.at[q], cnt_s.at[dst_b.at[q, ph]],
                                  ossem.at[q]).wait()

        issue_estage(_I0, 0)
        issue_estage(np.int32(1), 1)
        stage_compute(_I0, 0, 0)
        issue_gathers(0)
        stage_compute(np.int32(1), 1, 0)
        issue_gathers(1)

        def iter4(m, _):
            for ph in (0, 1):
                for q in (0, 1):
                    j = m * np.int32(4) + np.int32(2 * ph + q)
                    wait_gather(q)
                    pltpu.async_copy(rows_b.at[q], acc_s.at[dst_b.at[q, ph]],
                                     ssem.at[q], add=True)
                    if with_counts:
                        wait_ohgather(q)
                        pltpu.async_copy(oh_b.at[q],
                                         cnt_s.at[dst_b.at[q, ph]],
                                         ossem.at[q], add=True)
                    stage_compute(j + np.int32(2), q, ph ^ 1)
                    wait_scatter(q, ph)
                    if with_counts:
                        wait_ohscatter(q, ph)
                    issue_gathers(q)
            return _I0
        lax.fori_loop(_I0, np.int32(NCH // 4), iter4, _I0)

        for q in (0, 1):
            wait_gather(q)
            pltpu.make_async_copy(edges_hbm.at[pl.ds(_I0, 1)], ebuf.at[q],
                                  esem.at[q]).wait()
            if with_counts:
                wait_ohgather(q)

        plsc.subcore_barrier()

        pltpu.sync_copy(
            acc_s.at[pl.ds(rbase, ROWS_PER_TILE)],
            acc_out.at[c, pl.ds(rbase, ROWS_PER_TILE)])
        if with_counts:
            pltpu.sync_copy(
                cnt_s.at[pl.ds(rbase, ROWS_PER_TILE)],
                cnt_out.at[c, pl.ds(rbase, ROWS_PER_TILE)])

    return pl.kernel(
        body,
        out_type=tuple(out_type) if with_counts else out_type[0],
        mesh=mesh,
        compiler_params=pltpu.CompilerParams(
            needs_layout_passes=False, use_tc_tiling_on_sc=False),
        scratch_types=scratch,
    )


_sc_pass_counts = _make_sc_pass(True)
_sc_pass_plain = _make_sc_pass(False)


def _make_dense(with_emb):
    def body(*refs):
        if with_emb:
            x_ref, ew_ref, eb_ref, bs_ref, cf_ref, lp_ref, proj_ref, hl_ref = refs
            h = jnp.dot(x_ref[...], ew_ref[...],
                        preferred_element_type=jnp.float32) + eb_ref[...]
        else:
            x_ref, bs_ref, cf_ref, lp_ref, proj_ref, hl_ref = refs
            h = x_ref[...]
        hl_ref[...] = jnp.dot(h, lp_ref[...], preferred_element_type=jnp.float32)
        for b in range(NB):
            t = jnp.dot(h, bs_ref[b], preferred_element_type=jnp.float32)
            for r in range(R):
                contrib = cf_ref[r, b] * t
                if b == 0:
                    proj_ref[r] = contrib
                else:
                    proj_ref[r] = proj_ref[r] + contrib

    in_specs = [pl.BlockSpec((BN_BLK, D), lambda i: (i, 0))]
    if with_emb:
        in_specs += [
            pl.BlockSpec((D, D), lambda i: (0, 0)),
            pl.BlockSpec((1, D), lambda i: (0, 0)),
        ]
    in_specs += [
        pl.BlockSpec((NB, D, D), lambda i: (0, 0, 0)),
        pl.BlockSpec(memory_space=pltpu.SMEM),
        pl.BlockSpec((D, D), lambda i: (0, 0)),
    ]
    return pl.pallas_call(
        body,
        grid=(GRID,),
        in_specs=in_specs,
        out_specs=(
            pl.BlockSpec((R, BN_BLK, D), lambda i: (0, i, 0)),
            pl.BlockSpec((BN_BLK, D), lambda i: (i, 0)),
        ),
        out_shape=(
            jax.ShapeDtypeStruct((R, N, D), jnp.float32),
            jax.ShapeDtypeStruct((N, D), jnp.float32),
        ),
    )


_dense_emb = _make_dense(True)
_dense_plain = _make_dense(False)


def _bn_relu_block(acc_ref, cnt_ref, hl_ref, g_ref, b_ref):
    p = acc_ref[0] + acc_ref[1]
    cn = cnt_ref[0] + cnt_ref[1]
    norm = jnp.zeros((BN_BLK, 1), jnp.float32)
    for r_ in range(R):
        cr = cn[:, r_:r_ + 1]
        norm = jnp.where(cr > 0, 1.0 / cr, norm)
    o = p * norm + hl_ref[...]
    o = g_ref[...] * (o * INV_BN) + b_ref[...]
    return jnp.maximum(o, 0.0)


def _finalize_body(acc_ref, cnt_ref, hl_ref, g_ref, b_ref, o_ref):
    o_ref[...] = _bn_relu_block(acc_ref, cnt_ref, hl_ref, g_ref, b_ref)


def _fused_body(acc_ref, cnt_ref, hl_ref, g_ref, b_ref, bs_ref, cf_ref,
                lp_ref, proj_ref, hl2_ref):
    h = _bn_relu_block(acc_ref, cnt_ref, hl_ref, g_ref, b_ref)
    hl2_ref[...] = jnp.dot(h, lp_ref[...], preferred_element_type=jnp.float32)
    for b in range(NB):
        t = jnp.dot(h, bs_ref[b], preferred_element_type=jnp.float32)
        for r in range(R):
            contrib = cf_ref[r, b] * t
            if b == 0:
                proj_ref[r] = contrib
            else:
                proj_ref[r] = proj_ref[r] + contrib


_fused_fin_dense = pl.pallas_call(
    _fused_body,
    grid=(GRID,),
    in_specs=[
        pl.BlockSpec((NCORES, BN_BLK, D), lambda i: (0, i, 0)),
        pl.BlockSpec((NCORES, BN_BLK, CW), lambda i: (0, i, 0)),
        pl.BlockSpec((BN_BLK, D), lambda i: (i, 0)),
        pl.BlockSpec((1, D), lambda i: (0, 0)),
        pl.BlockSpec((1, D), lambda i: (0, 0)),
        pl.BlockSpec((NB, D, D), lambda i: (0, 0, 0)),
        pl.BlockSpec(memory_space=pltpu.SMEM),
        pl.BlockSpec((D, D), lambda i: (0, 0)),
    ],
    out_specs=(
        pl.BlockSpec((R, BN_BLK, D), lambda i: (0, i, 0)),
        pl.BlockSpec((BN_BLK, D), lambda i: (i, 0)),
    ),
    out_shape=(
        jax.ShapeDtypeStruct((R, N, D), jnp.float32),
        jax.ShapeDtypeStruct((N, D), jnp.float32),
    ),
)


_finalize = pl.pallas_call(
    _finalize_body,
    grid=(GRID,),
    in_specs=[
        pl.BlockSpec((NCORES, BN_BLK, D), lambda i: (0, i, 0)),
        pl.BlockSpec((NCORES, BN_BLK, CW), lambda i: (0, i, 0)),
        pl.BlockSpec((BN_BLK, D), lambda i: (i, 0)),
        pl.BlockSpec((1, D), lambda i: (0, 0)),
        pl.BlockSpec((1, D), lambda i: (0, 0)),
    ],
    out_specs=pl.BlockSpec((BN_BLK, D), lambda i: (i, 0)),
    out_shape=jax.ShapeDtypeStruct((N, D), jnp.float32),
)


def kernel(x, edge_index, edge_type, emb_w, emb_b, basis0, coef0, loop0,
           gamma0, beta0, basis1, coef1, loop1, gamma1, beta1):
    with jax.enable_x64(False):
        return _kernel_32(x, edge_index, edge_type, emb_w, emb_b, basis0,
                          coef0, loop0, gamma0, beta0, basis1, coef1, loop1,
                          gamma1, beta1)


def _kernel_32(x, edge_index, edge_type, emb_w, emb_b, basis0, coef0, loop0,
               gamma0, beta0, basis1, coef1, loop1, gamma1, beta1):
    src = edge_index[0].astype(jnp.int32)
    dst = edge_index[1].astype(jnp.int32)
    typ = edge_type.astype(jnp.int32)

    npad = E_PAD - E
    ar = jnp.arange(npad, dtype=jnp.int32)
    src_p = jnp.concatenate([src, ar % N]).reshape(E_PAD // CK, CK)
    dst_p = jnp.concatenate([dst, N + ar % (N_ACC - N)]).reshape(E_PAD // CK, CK)
    typ_p = jnp.concatenate([typ, jnp.zeros((npad,), jnp.int32)]).reshape(
        E_PAD // CK, CK)
    edges = jnp.concatenate(
        [jnp.stack([src_p, dst_p, typ_p], axis=1),
         jnp.zeros((4, 3, CK), jnp.int32)], axis=0)

    emb_b2 = emb_b.reshape(1, D)
    g0 = gamma0.reshape(1, D)
    b0 = beta0.reshape(1, D)
    g1 = gamma1.reshape(1, D)
    b1 = beta1.reshape(1, D)

    proj1, hl1 = _dense_emb(x, emb_w, emb_b2, basis0, coef0, loop0)
    acc1, cnt = _sc_pass_counts(proj1.reshape(R * N, D), edges,
                                jnp.asarray(_OHTAB))
    proj2, hl2 = _fused_fin_dense(acc1, cnt, hl1, g0, b0, basis1, coef1, loop1)
    acc2 = _sc_pass_plain(proj2.reshape(R * N, D), edges)
    h2 = _finalize(acc2, cnt, hl2, g1, b1)
    return h2


# speedup vs baseline: 46.3598x; 1.0034x over previous
import math

import numpy as np

import jax
import jax.numpy as jnp
from jax import lax
from jax.experimental import pallas as pl
from jax.experimental.pallas import tpu as pltpu
from jax.experimental.pallas import tpu_sc as plsc

N = 10000
D = 128
R = 8
NB = 4
E = 320000
EPS = 1e-3
INV_BN = float(1.0 / math.sqrt(1.0 + EPS))

NCORES = 2
NSUB = 16
NTILES = NCORES * NSUB
CK = 128
NCH = 80
ET = NCH * CK
E_PAD = NTILES * ET
N_ACC = 10240
ROWS_PER_TILE = N_ACC // NSUB
CW = 16
BN_BLK = 2000
GRID = N // BN_BLK

_I0 = np.int32(0)
_ICK = np.int32(CK)
_ICH = np.int32(NCH)
_IN = np.int32(N)
_INSUB = np.int32(NSUB)
_IRPT = np.int32(ROWS_PER_TILE)
_I32F = np.int32(32)
_I31 = np.int32(31)

_OHTAB = np.repeat(np.eye(CW, dtype=np.float32)[:R], 32, axis=0)


def _make_sc_pass(with_counts):
    mesh = plsc.VectorSubcoreMesh(core_axis_name="c", subcore_axis_name="s")
    out_type = [jax.ShapeDtypeStruct((NCORES, N_ACC, D), jnp.float32)]
    if with_counts:
        out_type.append(jax.ShapeDtypeStruct((NCORES, N_ACC, CW), jnp.float32))
    scratch = [
        pltpu.VMEM((2, 1, 3, CK), jnp.int32),
        pltpu.VMEM((2, 2, CK), jnp.int32),
        pltpu.VMEM((2, CK), jnp.int32),
        pltpu.VMEM((2, CK, D), jnp.float32),
        pltpu.VMEM_SHARED((N_ACC, D), jnp.float32),
        pltpu.SemaphoreType.DMA((2,)),
        pltpu.SemaphoreType.DMA((2,)),
        pltpu.SemaphoreType.DMA((2,)),
    ]
    if with_counts:
        scratch += [
            pltpu.VMEM((2, CK), jnp.int32),
            pltpu.VMEM((2, CK, CW), jnp.float32),
            pltpu.VMEM_SHARED((N_ACC, CW), jnp.float32),
            pltpu.VMEM_SHARED((R * 32, CW), jnp.float32),
            pltpu.SemaphoreType.DMA((2,)),
            pltpu.SemaphoreType.DMA((2,)),
        ]

    def body(proj_hbm, edges_hbm, *rest):
        if with_counts:
            (ohtab_hbm, acc_out, cnt_out, ebuf, dst_b, key_b, rows_b, acc_s,
             gsem, ssem, esem, key2_b, oh_b, cnt_s, ohtab_s, ogsem,
             ossem) = rest
        else:
            acc_out, ebuf, dst_b, key_b, rows_b, acc_s, gsem, ssem, esem = rest
        c = lax.axis_index("c")
        s = lax.axis_index("s")
        wid = c * _INSUB + s
        rbase = s * _IRPT

        def zrow(i, _):
            for v in range(D // 16):
                rows_b[0, i, pl.ds(v * 16, 16)] = jnp.zeros((16,), jnp.float32)
            return _I0
        lax.fori_loop(_I0, _ICK, zrow, _I0)
        for blk in range(ROWS_PER_TILE // CK):
            pltpu.sync_copy(rows_b.at[0],
                            acc_s.at[pl.ds(rbase + np.int32(blk * CK), CK)])
        if with_counts:
            def zoh(i, _):
                oh_b[0, i, pl.ds(0, CW)] = jnp.zeros((CW,), jnp.float32)
                return _I0
            lax.fori_loop(_I0, _ICK, zoh, _I0)
            for blk in range(ROWS_PER_TILE // CK):
                pltpu.sync_copy(
                    oh_b.at[0], cnt_s.at[pl.ds(rbase + np.int32(blk * CK), CK)])
            pltpu.sync_copy(ohtab_hbm, ohtab_s)

        plsc.subcore_barrier()

        def issue_estage(j, q):
            row = wid * _ICH + j
            pltpu.async_copy(edges_hbm.at[pl.ds(row, 1)], ebuf.at[q],
                             esem.at[q])

        def stage_compute(j, q, ph):
            pltpu.make_async_copy(edges_hbm.at[pl.ds(_I0, 1)], ebuf.at[q],
                                  esem.at[q]).wait()
            for v in range(CK // 16):
                sv = ebuf[q, 0, 0, pl.ds(v * 16, 16)]
                dv = ebuf[q, 0, 1, pl.ds(v * 16, 16)]
                tv = ebuf[q, 0, 2, pl.ds(v * 16, 16)]
                key_b[q, pl.ds(v * 16, 16)] = tv * _IN + sv
                dst_b[q, ph, pl.ds(v * 16, 16)] = dv
                if with_counts:
                    key2_b[q, pl.ds(v * 16, 16)] = tv * _I32F + (sv & _I31)
            issue_estage(j + np.int32(2), q)

        def issue_gathers(q):
            pltpu.async_copy(proj_hbm.at[key_b.at[q]], rows_b.at[q], gsem.at[q])
            if with_counts:
                pltpu.async_copy(ohtab_s.at[key2_b.at[q]], oh_b.at[q],
                                 ogsem.at[q])

        def wait_gather(q):
            pltpu.make_async_copy(proj_hbm.at[key_b.at[q]], rows_b.at[q],
                                  gsem.at[q]).wait()

        def wait_scatter(q, ph):
            pltpu.make_async_copy(rows_b.at[q], acc_s.at[dst_b.at[q, ph]],
                                  ssem.at[q]).wait()

        def wait_ohgather(q):
            pltpu.make_async_copy(ohtab_s.at[key2_b.at[q]], oh_b.at[q],
                                  ogsem.at[q]).wait()

        def wait_ohscatter(q, ph):
            pltpu.make_async_copy(oh_b.at[q], cnt_s.at[dst_b.at[q, ph]],
                                  ossem.at[q]).wait()

        issue_estage(_I0, 0)
        issue_estage(np.int32(1), 1)
        stage_compute(_I0, 0, 0)
        issue_gathers(0)
        stage_compute(np.int32(1), 1, 0)
        issue_gathers(1)

        def iter4(m, _):
            for ph in (0, 1):
                for q in (0, 1):
                    j = m * np.int32(4) + np.int32(2 * ph + q)
                    wait_gather(q)
                    pltpu.async_copy(rows_b.at[q], acc_s.at[dst_b.at[q, ph]],
                                     ssem.at[q], add=True)
                    if with_counts:
                        wait_ohgather(q)
                        pltpu.async_copy(oh_b.at[q],
                                         cnt_s.at[dst_b.at[q, ph]],
                                         ossem.at[q], add=True)
                    stage_compute(j + np.int32(2), q, ph ^ 1)
                    wait_scatter(q, ph)
                    if with_counts:
                        wait_ohscatter(q, ph)
                    issue_gathers(q)
            return _I0
        lax.fori_loop(_I0, np.int32(NCH // 4), iter4, _I0)

        for q in (0, 1):
            wait_gather(q)
            pltpu.make_async_copy(edges_hbm.at[pl.ds(_I0, 1)], ebuf.at[q],
                                  esem.at[q]).wait()
            if with_counts:
                wait_ohgather(q)

        plsc.subcore_barrier()

        pltpu.sync_copy(
            acc_s.at[pl.ds(rbase, ROWS_PER_TILE)],
            acc_out.at[c, pl.ds(rbase, ROWS_PER_TILE)])
        if with_counts:
            pltpu.sync_copy(
                cnt_s.at[pl.ds(rbase, ROWS_PER_TILE)],
                cnt_out.at[c, pl.ds(rbase, ROWS_PER_TILE)])

    return pl.kernel(
        body,
        out_type=tuple(out_type) if with_counts else out_type[0],
        mesh=mesh,
        compiler_params=pltpu.CompilerParams(
            needs_layout_passes=False, use_tc_tiling_on_sc=False),
        scratch_types=scratch,
    )


_sc_pass_counts = _make_sc_pass(True)
_sc_pass_plain = _make_sc_pass(False)


def _make_dense(with_emb):
    def body(*refs):
        if with_emb:
            x_ref, ew_ref, eb_ref, bs_ref, cf_ref, lp_ref, proj_ref, hl_ref = refs
            h = jnp.dot(x_ref[...], ew_ref[...],
                        preferred_element_type=jnp.float32) + eb_ref[...]
        else:
            x_ref, bs_ref, cf_ref, lp_ref, proj_ref, hl_ref = refs
            h = x_ref[...]
        hl_ref[...] = jnp.dot(h, lp_ref[...], preferred_element_type=jnp.float32)
        for b in range(NB):
            t = jnp.dot(h, bs_ref[b], preferred_element_type=jnp.float32)
            for r in range(R):
                contrib = cf_ref[r, b] * t
                if b == 0:
                    proj_ref[r] = contrib
                else:
                    proj_ref[r] = proj_ref[r] + contrib

    in_specs = [pl.BlockSpec((BN_BLK, D), lambda i: (i, 0))]
    if with_emb:
        in_specs += [
            pl.BlockSpec((D, D), lambda i: (0, 0)),
            pl.BlockSpec((1, D), lambda i: (0, 0)),
        ]
    in_specs += [
        pl.BlockSpec((NB, D, D), lambda i: (0, 0, 0)),
        pl.BlockSpec(memory_space=pltpu.SMEM),
        pl.BlockSpec((D, D), lambda i: (0, 0)),
    ]
    return pl.pallas_call(
        body,
        grid=(GRID,),
        in_specs=in_specs,
        out_specs=(
            pl.BlockSpec((R, BN_BLK, D), lambda i: (0, i, 0)),
            pl.BlockSpec((BN_BLK, D), lambda i: (i, 0)),
        ),
        out_shape=(
            jax.ShapeDtypeStruct((R, N, D), jnp.float32),
            jax.ShapeDtypeStruct((N, D), jnp.float32),
        ),
    )


_dense_emb = _make_dense(True)


def _bn_relu_block(acc_ref, cnt_ref, hl_ref, g_ref, b_ref):
    p = acc_ref[0] + acc_ref[1]
    cn = cnt_ref[0] + cnt_ref[1]
    norm = jnp.zeros((BN_BLK, 1), jnp.float32)
    for r_ in range(R):
        cr = cn[:, r_:r_ + 1]
        norm = jnp.where(cr > 0, 1.0 / cr, norm)
    o = p * norm + hl_ref[...]
    o = g_ref[...] * (o * INV_BN) + b_ref[...]
    return jnp.maximum(o, 0.0)


def _finalize_body(acc_ref, cnt_ref, hl_ref, g_ref, b_ref, o_ref):
    o_ref[...] = _bn_relu_block(acc_ref, cnt_ref, hl_ref, g_ref, b_ref)


def _fused_body(acc_ref, cnt_ref, hl_ref, g_ref, b_ref, bs_ref, cf_ref,
                lp_ref, proj_ref, hl2_ref):
    h = _bn_relu_block(acc_ref, cnt_ref, hl_ref, g_ref, b_ref)
    hl2_ref[...] = jnp.dot(h, lp_ref[...], preferred_element_type=jnp.float32)
    for b in range(NB):
        t = jnp.dot(h, bs_ref[b], preferred_element_type=jnp.float32)
        for r in range(R):
            contrib = cf_ref[r, b] * t
            if b == 0:
                proj_ref[r] = contrib
            else:
                proj_ref[r] = proj_ref[r] + contrib


_fused_fin_dense = pl.pallas_call(
    _fused_body,
    grid=(GRID,),
    in_specs=[
        pl.BlockSpec((NCORES, BN_BLK, D), lambda i: (0, i, 0)),
        pl.BlockSpec((NCORES, BN_BLK, CW), lambda i: (0, i, 0)),
        pl.BlockSpec((BN_BLK, D), lambda i: (i, 0)),
        pl.BlockSpec((1, D), lambda i: (0, 0)),
        pl.BlockSpec((1, D), lambda i: (0, 0)),
        pl.BlockSpec((NB, D, D), lambda i: (0, 0, 0)),
        pl.BlockSpec(memory_space=pltpu.SMEM),
        pl.BlockSpec((D, D), lambda i: (0, 0)),
    ],
    out_specs=(
        pl.BlockSpec((R, BN_BLK, D), lambda i: (0, i, 0)),
        pl.BlockSpec((BN_BLK, D), lambda i: (i, 0)),
    ),
    out_shape=(
        jax.ShapeDtypeStruct((R, N, D), jnp.float32),
        jax.ShapeDtypeStruct((N, D), jnp.float32),
    ),
)


_finalize = pl.pallas_call(
    _finalize_body,
    grid=(GRID,),
    in_specs=[
        pl.BlockSpec((NCORES, BN_BLK, D), lambda i: (0, i, 0)),
        pl.BlockSpec((NCORES, BN_BLK, CW), lambda i: (0, i, 0)),
        pl.BlockSpec((BN_BLK, D), lambda i: (i, 0)),
        pl.BlockSpec((1, D), lambda i: (0, 0)),
        pl.BlockSpec((1, D), lambda i: (0, 0)),
    ],
    out_specs=pl.BlockSpec((BN_BLK, D), lambda i: (i, 0)),
    out_shape=jax.ShapeDtypeStruct((N, D), jnp.float32),
)


def kernel(x, edge_index, edge_type, emb_w, emb_b, basis0, coef0, loop0,
           gamma0, beta0, basis1, coef1, loop1, gamma1, beta1):
    with jax.enable_x64(False):
        return _kernel_32(x, edge_index, edge_type, emb_w, emb_b, basis0,
                          coef0, loop0, gamma0, beta0, basis1, coef1, loop1,
                          gamma1, beta1)


def _kernel_32(x, edge_index, edge_type, emb_w, emb_b, basis0, coef0, loop0,
               gamma0, beta0, basis1, coef1, loop1, gamma1, beta1):
    src = edge_index[0].astype(jnp.int32)
    dst = edge_index[1].astype(jnp.int32)
    typ = edge_type.astype(jnp.int32)

    npad = E_PAD - E
    ar = jnp.arange(npad, dtype=jnp.int32)
    src_p = jnp.concatenate([src, ar % N]).reshape(E_PAD // CK, CK)
    dst_p = jnp.concatenate([dst, N + ar % (N_ACC - N)]).reshape(E_PAD // CK, CK)
    typ_p = jnp.concatenate([typ, jnp.zeros((npad,), jnp.int32)]).reshape(
        E_PAD // CK, CK)
    edges = jnp.concatenate(
        [jnp.stack([src_p, dst_p, typ_p], axis=1),
         jnp.zeros((4, 3, CK), jnp.int32)], axis=0)

    emb_b2 = emb_b.reshape(1, D)
    g0 = gamma0.reshape(1, D)
    b0 = beta0.reshape(1, D)
    g1 = gamma1.reshape(1, D)
    b1 = beta1.reshape(1, D)

    proj1, hl1 = _dense_emb(x, emb_w, emb_b2, basis0, coef0, loop0)
    acc1, cnt = _sc_pass_counts(proj1.reshape(R * N, D), edges,
                                jnp.asarray(_OHTAB))
    proj2, hl2 = _fused_fin_dense(acc1, cnt, hl1, g0, b0, basis1, coef1, loop1)
    acc2 = _sc_pass_plain(proj2.reshape(R * N, D), edges)
    h2 = _finalize(acc2, cnt, hl2, g1, b1)
    return h2
